# Initial kernel scaffold; baseline (speedup 1.0000x reference)
import jax
import jax.numpy as jnp
from jax.experimental import pallas as pl


def kernel(checkin_feature, delta_ts, delta_ss, ci2traj_attr, traj2traj_attr, time_w, time_b, dist_w, dist_b, tw, sw, ln_g, ln_b, W_out, b_out, ci2traj_row, ci2traj_col, traj2traj_row, traj2traj_col):
    raise NotImplementedError("write your pallas kernel here")


# speedup vs baseline: 4.7471x; 4.7471x over previous
import functools

import jax
import jax.numpy as jnp
from jax import lax
from jax.experimental import pallas as pl
from jax.experimental.pallas import tpu as pltpu
from jax.experimental.pallas import tpu_sc as plsc

_NUM_TRAJ = 2000
_NUM_CHECKIN = 8000
_N = _NUM_TRAJ + _NUM_CHECKIN
_D = 128
_E1 = 160000
_NUM_POI = 5000

_NC = 2
_NS = 16
_NW = _NC * _NS
_CHUNK = 128
_NCHUNK = 40
_EPT = _CHUNK * _NCHUNK
_E_PAD = _EPT * _NW
_SEG_PAD = 8192
_RPT = _SEG_PAD // _NS


def _fusion_body(dts_ref, dss_ref, ci_ref, timew_ref, timeb_ref, distw_ref,
                 distb_ref, tw_ref, sw_ref, xtraj_ref, xci_ref, fuse_ref):
    t_emb = jnp.cos(dts_ref[...] * (1.0 / 3600.0) * timew_ref[...]
                    + timeb_ref[...])
    s_emb = jnp.maximum(dss_ref[...] * distw_ref[...] + distb_ref[...], 0.0)
    fuse = tw_ref[...] * t_emb + sw_ref[...] * s_emb
    xtraj_ref[...] = jnp.maximum(fuse[:_NUM_TRAJ], 0.0)
    xci_ref[...] = jnp.maximum(ci_ref[...] + fuse[_NUM_TRAJ:], 0.0)
    fuse_ref[...] = fuse[_NUM_TRAJ:]


_fusion_call = pl.pallas_call(
    _fusion_body,
    out_shape=(
        jax.ShapeDtypeStruct((_NUM_TRAJ, _D), jnp.float32),
        jax.ShapeDtypeStruct((_NUM_CHECKIN, _D), jnp.float32),
        jax.ShapeDtypeStruct((_NUM_CHECKIN, _D), jnp.float32),
    ),
)


def _sc_edge_body(xtraj, rowr, colr, attrr, agg_out, ss_out,
                  agg_sh, ss_sh, row_v, col_v, attr_v, rows_v, exp_v,
                  zrow_v, sem):
    c = lax.axis_index("c")
    s = lax.axis_index("s")
    wid = c * _NS + s

    zero16 = jnp.zeros((16,), jnp.float32)

    def _zrow(i, carry):
        for k in range(8):
            rows_v[i, pl.ds(k * 16, 16)] = zero16
        return carry
    lax.fori_loop(0, 128, _zrow, 0)
    for k in range(32):
        zrow_v[pl.ds(k * 16, 16)] = zero16
    for j in range(_RPT // 128):
        pltpu.sync_copy(rows_v, agg_sh.at[pl.ds(s * _RPT + j * 128, 128)])
    pltpu.sync_copy(zrow_v, ss_sh.at[pl.ds(s * _RPT, _RPT)])

    pltpu.sync_copy(rowr.at[pl.ds(wid * _NCHUNK, _NCHUNK)], row_v)
    pltpu.sync_copy(colr.at[pl.ds(wid * _NCHUNK, _NCHUNK)], col_v)
    pltpu.sync_copy(attrr.at[pl.ds(wid * _NCHUNK, _NCHUNK)], attr_v)

    plsc.subcore_barrier()

    def _chunk(j, carry):
        pltpu.async_copy(xtraj.at[col_v.at[j]], rows_v, sem).wait()

        def _group(g, carry2):
            a = attr_v[j, pl.ds(g * 16, 16)]
            w = jnp.exp(a)
            exp_v[pl.ds(g * 16, 16)] = w
            for t in range(16):
                ws = jnp.full((16,), w[t])
                e = g * 16 + t
                for k in range(8):
                    sl = pl.ds(k * 16, 16)
                    rows_v[e, sl] = rows_v[e, sl] * ws
            return carry2
        lax.fori_loop(0, 8, _group, 0)

        pltpu.sync_copy(rows_v, agg_sh.at[row_v.at[j]], add=True)
        pltpu.sync_copy(exp_v, ss_sh.at[row_v.at[j]], add=True)
        return carry
    lax.fori_loop(0, _NCHUNK, _chunk, 0)

    plsc.subcore_barrier()

    pltpu.sync_copy(agg_sh.at[pl.ds(s * _RPT, _RPT)],
                    agg_out.at[c, pl.ds(s * _RPT, _RPT)])
    pltpu.sync_copy(ss_sh.at[pl.ds(s * _RPT, _RPT)],
                    ss_out.at[c, pl.ds(s * _RPT, _RPT)])


_sc_edge_call = pl.kernel(
    _sc_edge_body,
    out_type=(
        jax.ShapeDtypeStruct((_NC, _SEG_PAD, _D), jnp.float32),
        jax.ShapeDtypeStruct((_NC, _SEG_PAD), jnp.float32),
    ),
    mesh=plsc.VectorSubcoreMesh(core_axis_name="c", subcore_axis_name="s",
                                num_cores=_NC, num_subcores=_NS),
    scratch_types=[
        pltpu.VMEM_SHARED((_SEG_PAD, _D), jnp.float32),
        pltpu.VMEM_SHARED((_SEG_PAD,), jnp.float32),
        pltpu.VMEM((_NCHUNK, _CHUNK), jnp.int32),
        pltpu.VMEM((_NCHUNK, _CHUNK), jnp.int32),
        pltpu.VMEM((_NCHUNK, _CHUNK), jnp.float32),
        pltpu.VMEM((_CHUNK, _D), jnp.float32),
        pltpu.VMEM((_CHUNK,), jnp.float32),
        pltpu.VMEM((_RPT,), jnp.float32),
        pltpu.SemaphoreType.DMA,
    ],
)


_BM = 1000
_BN = 512
_NPOI_PAD = 5120


def _head_body(xci_ref, fuse_ref, agg0_ref, agg1_ref, ss0_ref, ss1_ref,
               g_ref, b_ref, w_ref, bias_ref, out_ref, h_scr):
    @pl.when(pl.program_id(1) == 0)
    def _():
        denom = ss0_ref[...] + ss1_ref[...] + 1e-30
        h = xci_ref[...] + (agg0_ref[...] + agg1_ref[...]) / denom
        mu = jnp.mean(h, axis=1, keepdims=True)
        var = jnp.mean((h - mu) * (h - mu), axis=1, keepdims=True)
        h = (h - mu) * jax.lax.rsqrt(var + 1e-5) * g_ref[...] + b_ref[...]
        h_scr[...] = h + fuse_ref[...]

    out_ref[...] = jnp.dot(h_scr[...], w_ref[...],
                           preferred_element_type=jnp.float32) + bias_ref[...]


_head_call = pl.pallas_call(
    _head_body,
    grid=(_NUM_CHECKIN // _BM, _NPOI_PAD // _BN),
    in_specs=[
        pl.BlockSpec((_BM, _D), lambda m, n: (m, 0)),
        pl.BlockSpec((_BM, _D), lambda m, n: (m, 0)),
        pl.BlockSpec((_BM, _D), lambda m, n: (m, 0)),
        pl.BlockSpec((_BM, _D), lambda m, n: (m, 0)),
        pl.BlockSpec((_BM, 1), lambda m, n: (m, 0)),
        pl.BlockSpec((_BM, 1), lambda m, n: (m, 0)),
        pl.BlockSpec((1, _D), lambda m, n: (0, 0)),
        pl.BlockSpec((1, _D), lambda m, n: (0, 0)),
        pl.BlockSpec((_D, _BN), lambda m, n: (0, n)),
        pl.BlockSpec((1, _BN), lambda m, n: (0, n)),
    ],
    out_specs=pl.BlockSpec((_BM, _BN), lambda m, n: (m, n)),
    out_shape=jax.ShapeDtypeStruct((_NUM_CHECKIN, _NPOI_PAD), jnp.float32),
    scratch_shapes=[pltpu.VMEM((_BM, _D), jnp.float32)],
)


def kernel(checkin_feature, delta_ts, delta_ss, ci2traj_attr, traj2traj_attr,
           time_w, time_b, dist_w, dist_b, tw, sw, ln_g, ln_b, W_out, b_out,
           ci2traj_row, ci2traj_col, traj2traj_row, traj2traj_col):
    del traj2traj_attr, traj2traj_row, traj2traj_col

    dts = delta_ts.reshape(_N, 1)
    dss = delta_ss.reshape(_N, 1)
    xtraj0, xci0, fuse_ci = _fusion_call(
        dts, dss, checkin_feature,
        time_w.reshape(1, _D), time_b.reshape(1, _D),
        dist_w.reshape(1, _D), dist_b.reshape(1, _D),
        tw.reshape(1, _D), sw.reshape(1, _D))

    pad = _E_PAD - _E1
    row_p = jnp.pad(ci2traj_row.astype(jnp.int32), (0, pad)).reshape(
        _NW * _NCHUNK, _CHUNK)
    col_p = jnp.pad(ci2traj_col.astype(jnp.int32), (0, pad)).reshape(
        _NW * _NCHUNK, _CHUNK)
    attr_p = jnp.pad(ci2traj_attr, (0, pad), constant_values=-1e30).reshape(
        _NW * _NCHUNK, _CHUNK)

    agg_pair, ss_pair = _sc_edge_call(xtraj0, row_p, col_p, attr_p)

    wp = jnp.pad(W_out, ((0, 0), (0, _NPOI_PAD - _NUM_POI)))
    bp = jnp.pad(b_out, (0, _NPOI_PAD - _NUM_POI)).reshape(1, _NPOI_PAD)

    out = _head_call(
        xci0, fuse_ci,
        agg_pair[0, :_NUM_CHECKIN], agg_pair[1, :_NUM_CHECKIN],
        ss_pair[0, :_NUM_CHECKIN, None], ss_pair[1, :_NUM_CHECKIN, None],
        ln_g.reshape(1, _D), ln_b.reshape(1, _D), wp, bp)
    return out[:, :_NUM_POI]


# speedup vs baseline: 8.0948x; 1.7052x over previous
import functools

import jax
import jax.numpy as jnp
from jax import lax
from jax.experimental import pallas as pl
from jax.experimental.pallas import tpu as pltpu
from jax.experimental.pallas import tpu_sc as plsc

_NUM_TRAJ = 2000
_NUM_CHECKIN = 8000
_N = _NUM_TRAJ + _NUM_CHECKIN
_D = 128
_E1 = 160000
_NUM_POI = 5000

_NC = 2
_NS = 16
_NW = _NC * _NS
_CHUNK = 128
_NCHUNK = 40
_EPT = _CHUNK * _NCHUNK
_E_PAD = _EPT * _NW
_SEG_PAD = 8192
_RPT = _SEG_PAD // _NS


def _fusion_body(dts_ref, dss_ref, ci_ref, timew_ref, timeb_ref, distw_ref,
                 distb_ref, tw_ref, sw_ref, xtraj_ref, xci_ref, fuse_ref):
    t_emb = jnp.cos(dts_ref[...] * (1.0 / 3600.0) * timew_ref[...]
                    + timeb_ref[...])
    s_emb = jnp.maximum(dss_ref[...] * distw_ref[...] + distb_ref[...], 0.0)
    fuse = tw_ref[...] * t_emb + sw_ref[...] * s_emb
    xtraj_ref[...] = jnp.maximum(fuse[:_NUM_TRAJ], 0.0)
    xci_ref[...] = jnp.maximum(ci_ref[...] + fuse[_NUM_TRAJ:], 0.0)
    fuse_ref[...] = fuse[_NUM_TRAJ:]


_fusion_call = pl.pallas_call(
    _fusion_body,
    out_shape=(
        jax.ShapeDtypeStruct((_NUM_TRAJ, _D), jnp.float32),
        jax.ShapeDtypeStruct((_NUM_CHECKIN, _D), jnp.float32),
        jax.ShapeDtypeStruct((_NUM_CHECKIN, _D), jnp.float32),
    ),
)


def _sc_edge_body(xtraj, rowr, colr, attrr, agg_out, ss_out,
                  agg_sh, ss_sh, row_v, col_v, attr_v, rows_v, exp_v,
                  zrow_v, sem):
    c = lax.axis_index("c")
    s = lax.axis_index("s")
    wid = c * _NS + s

    zero16 = jnp.zeros((16,), jnp.float32)

    def _zrow(i, carry):
        for k in range(8):
            rows_v[i, pl.ds(k * 16, 16)] = zero16
        return carry
    lax.fori_loop(0, 128, _zrow, 0)
    for k in range(32):
        zrow_v[pl.ds(k * 16, 16)] = zero16
    for j in range(_RPT // 128):
        pltpu.sync_copy(rows_v, agg_sh.at[pl.ds(s * _RPT + j * 128, 128)])
    pltpu.sync_copy(zrow_v, ss_sh.at[pl.ds(s * _RPT, _RPT)])

    pltpu.sync_copy(rowr.at[pl.ds(wid * _NCHUNK, _NCHUNK)], row_v)
    pltpu.sync_copy(colr.at[pl.ds(wid * _NCHUNK, _NCHUNK)], col_v)
    pltpu.sync_copy(attrr.at[pl.ds(wid * _NCHUNK, _NCHUNK)], attr_v)

    plsc.subcore_barrier()

    def _chunk(j, carry):
        pltpu.async_copy(xtraj.at[col_v.at[j]], rows_v, sem).wait()

        def _group(g, carry2):
            a = attr_v[j, pl.ds(g * 16, 16)]
            w = jnp.exp(a)
            exp_v[pl.ds(g * 16, 16)] = w
            for t in range(16):
                ws = jnp.full((16,), w[t])
                e = g * 16 + t
                for k in range(8):
                    sl = pl.ds(k * 16, 16)
                    rows_v[e, sl] = rows_v[e, sl] * ws
            return carry2
        lax.fori_loop(0, 8, _group, 0)

        pltpu.sync_copy(rows_v, agg_sh.at[row_v.at[j]], add=True)
        pltpu.sync_copy(exp_v, ss_sh.at[row_v.at[j]], add=True)
        return carry
    lax.fori_loop(0, _NCHUNK, _chunk, 0)

    plsc.subcore_barrier()

    pltpu.sync_copy(agg_sh.at[pl.ds(s * _RPT, _RPT)],
                    agg_out.at[c, pl.ds(s * _RPT, _RPT)])
    pltpu.sync_copy(ss_sh.at[pl.ds(s * _RPT, _RPT)],
                    ss_out.at[c, pl.ds(s * _RPT, _RPT)])


_sc_edge_call = pl.kernel(
    _sc_edge_body,
    out_type=(
        jax.ShapeDtypeStruct((_NC, _SEG_PAD, _D), jnp.float32),
        jax.ShapeDtypeStruct((_NC, _SEG_PAD), jnp.float32),
    ),
    mesh=plsc.VectorSubcoreMesh(core_axis_name="c", subcore_axis_name="s",
                                num_cores=_NC, num_subcores=_NS),
    scratch_types=[
        pltpu.VMEM_SHARED((_SEG_PAD, _D), jnp.float32),
        pltpu.VMEM_SHARED((_SEG_PAD,), jnp.float32),
        pltpu.VMEM((_NCHUNK, _CHUNK), jnp.int32),
        pltpu.VMEM((_NCHUNK, _CHUNK), jnp.int32),
        pltpu.VMEM((_NCHUNK, _CHUNK), jnp.float32),
        pltpu.VMEM((_CHUNK, _D), jnp.float32),
        pltpu.VMEM((_CHUNK,), jnp.float32),
        pltpu.VMEM((_RPT,), jnp.float32),
        pltpu.SemaphoreType.DMA,
    ],
)


_BM = 1000
_BN = 512
_NPOI_PAD = 5120


def _head_body(xci_ref, fuse_ref, agg0_ref, agg1_ref, ss0_ref, ss1_ref,
               g_ref, b_ref, w_ref, bias_ref, out_ref, h_scr):
    @pl.when(pl.program_id(1) == 0)
    def _():
        denom = ss0_ref[...] + ss1_ref[...] + 1e-30
        h = xci_ref[...] + (agg0_ref[...] + agg1_ref[...]) / denom
        mu = jnp.mean(h, axis=1, keepdims=True)
        var = jnp.mean((h - mu) * (h - mu), axis=1, keepdims=True)
        h = (h - mu) * jax.lax.rsqrt(var + 1e-5) * g_ref[...] + b_ref[...]
        h_scr[...] = h + fuse_ref[...]

    out_ref[...] = jnp.dot(h_scr[...], w_ref[...],
                           preferred_element_type=jnp.float32) + bias_ref[...]


_head_call = pl.pallas_call(
    _head_body,
    grid=(_NUM_CHECKIN // _BM, _NPOI_PAD // _BN),
    in_specs=[
        pl.BlockSpec((_BM, _D), lambda m, n: (m, 0)),
        pl.BlockSpec((_BM, _D), lambda m, n: (m, 0)),
        pl.BlockSpec((_BM, _D), lambda m, n: (m, 0)),
        pl.BlockSpec((_BM, _D), lambda m, n: (m, 0)),
        pl.BlockSpec((_BM, 1), lambda m, n: (m, 0)),
        pl.BlockSpec((_BM, 1), lambda m, n: (m, 0)),
        pl.BlockSpec((1, _D), lambda m, n: (0, 0)),
        pl.BlockSpec((1, _D), lambda m, n: (0, 0)),
        pl.BlockSpec((_D, _BN), lambda m, n: (0, n)),
        pl.BlockSpec((1, _BN), lambda m, n: (0, n)),
    ],
    out_specs=pl.BlockSpec((_BM, _BN), lambda m, n: (m, n)),
    out_shape=jax.ShapeDtypeStruct((_NUM_CHECKIN, _NUM_POI), jnp.float32),
    scratch_shapes=[pltpu.VMEM((_BM, _D), jnp.float32)],
)


def kernel(checkin_feature, delta_ts, delta_ss, ci2traj_attr, traj2traj_attr,
           time_w, time_b, dist_w, dist_b, tw, sw, ln_g, ln_b, W_out, b_out,
           ci2traj_row, ci2traj_col, traj2traj_row, traj2traj_col):
    del traj2traj_attr, traj2traj_row, traj2traj_col

    dts = delta_ts.reshape(_N, 1)
    dss = delta_ss.reshape(_N, 1)
    xtraj0, xci0, fuse_ci = _fusion_call(
        dts, dss, checkin_feature,
        time_w.reshape(1, _D), time_b.reshape(1, _D),
        dist_w.reshape(1, _D), dist_b.reshape(1, _D),
        tw.reshape(1, _D), sw.reshape(1, _D))

    pad = _E_PAD - _E1
    row_p = jnp.pad(ci2traj_row.astype(jnp.int32), (0, pad)).reshape(
        _NW * _NCHUNK, _CHUNK)
    col_p = jnp.pad(ci2traj_col.astype(jnp.int32), (0, pad)).reshape(
        _NW * _NCHUNK, _CHUNK)
    attr_p = jnp.pad(ci2traj_attr, (0, pad), constant_values=-1e30).reshape(
        _NW * _NCHUNK, _CHUNK)

    agg_pair, ss_pair = _sc_edge_call(xtraj0, row_p, col_p, attr_p)

    wp = jnp.pad(W_out, ((0, 0), (0, _NPOI_PAD - _NUM_POI)))
    bp = jnp.pad(b_out, (0, _NPOI_PAD - _NUM_POI)).reshape(1, _NPOI_PAD)

    return _head_call(
        xci0, fuse_ci,
        agg_pair[0, :_NUM_CHECKIN], agg_pair[1, :_NUM_CHECKIN],
        ss_pair[0, :_NUM_CHECKIN, None], ss_pair[1, :_NUM_CHECKIN, None],
        ln_g.reshape(1, _D), ln_b.reshape(1, _D), wp, bp)


# speedup vs baseline: 8.7325x; 1.0788x over previous
import functools

import jax
import jax.numpy as jnp
from jax import lax
from jax.experimental import pallas as pl
from jax.experimental.pallas import tpu as pltpu
from jax.experimental.pallas import tpu_sc as plsc

_NUM_TRAJ = 2000
_NUM_CHECKIN = 8000
_N = _NUM_TRAJ + _NUM_CHECKIN
_D = 128
_E1 = 160000
_NUM_POI = 5000

_NC = 2
_NS = 16
_NW = _NC * _NS
_CHUNK = 128
_NCHUNK = 40
_EPT = _CHUNK * _NCHUNK
_E_PAD = _EPT * _NW
_SEG_PAD = 8192
_RPT = _SEG_PAD // _NS


def _fusion_body(dts_ref, dss_ref, ci_ref, timew_ref, timeb_ref, distw_ref,
                 distb_ref, tw_ref, sw_ref, xtraj_ref, xci_ref, fuse_ref):
    t_emb = jnp.cos(dts_ref[...] * (1.0 / 3600.0) * timew_ref[...]
                    + timeb_ref[...])
    s_emb = jnp.maximum(dss_ref[...] * distw_ref[...] + distb_ref[...], 0.0)
    fuse = tw_ref[...] * t_emb + sw_ref[...] * s_emb
    xtraj_ref[...] = jnp.maximum(fuse[:_NUM_TRAJ], 0.0)
    xci_ref[...] = jnp.maximum(ci_ref[...] + fuse[_NUM_TRAJ:], 0.0)
    fuse_ref[...] = fuse[_NUM_TRAJ:]


_fusion_call = pl.pallas_call(
    _fusion_body,
    out_shape=(
        jax.ShapeDtypeStruct((_NUM_TRAJ, _D), jnp.float32),
        jax.ShapeDtypeStruct((_NUM_CHECKIN, _D), jnp.float32),
        jax.ShapeDtypeStruct((_NUM_CHECKIN, _D), jnp.float32),
    ),
)


def _sc_edge_body(xtraj, rowr, colr, attrr, agg_out, ss_out,
                  agg_sh, ss_sh, row_v, col_v, attr_v, rows_a, rows_b, exp_v,
                  zrow_v, sem_a, sem_b):
    c = lax.axis_index("c")
    s = lax.axis_index("s")
    wid = c * _NS + s

    zero16 = jnp.zeros((16,), jnp.float32)

    def _zrow(i, carry):
        for k in range(8):
            rows_a[i, pl.ds(k * 16, 16)] = zero16
        return carry
    lax.fori_loop(0, 128, _zrow, 0)
    for k in range(32):
        zrow_v[pl.ds(k * 16, 16)] = zero16
    for j in range(_RPT // 128):
        pltpu.sync_copy(rows_a, agg_sh.at[pl.ds(s * _RPT + j * 128, 128)])
    pltpu.sync_copy(zrow_v, ss_sh.at[pl.ds(s * _RPT, _RPT)])

    pltpu.sync_copy(rowr.at[pl.ds(wid * _NCHUNK, _NCHUNK)], row_v)
    pltpu.sync_copy(colr.at[pl.ds(wid * _NCHUNK, _NCHUNK)], col_v)
    pltpu.sync_copy(attrr.at[pl.ds(wid * _NCHUNK, _NCHUNK)], attr_v)

    plsc.subcore_barrier()

    def _process(j, buf):
        def _group(g, carry2):
            a = attr_v[j, pl.ds(g * 16, 16)]
            w = jnp.exp(a)
            exp_v[pl.ds(g * 16, 16)] = w
            for t in range(16):
                ws = jnp.full((16,), w[t])
                e = g * 16 + t
                for k in range(8):
                    sl = pl.ds(k * 16, 16)
                    buf[e, sl] = buf[e, sl] * ws
            return carry2
        lax.fori_loop(0, 8, _group, 0)

        pltpu.sync_copy(buf, agg_sh.at[row_v.at[j]], add=True)
        pltpu.sync_copy(exp_v, ss_sh.at[row_v.at[j]], add=True)

    pltpu.async_copy(xtraj.at[col_v.at[0]], rows_a, sem_a)

    def _pair(p, carry):
        c0 = 2 * p
        pltpu.make_async_copy(xtraj.at[col_v.at[c0]], rows_a, sem_a).wait()
        pltpu.async_copy(xtraj.at[col_v.at[c0 + 1]], rows_b, sem_b)
        _process(c0, rows_a)
        pltpu.make_async_copy(xtraj.at[col_v.at[c0 + 1]], rows_b, sem_b).wait()
        c2 = jnp.minimum(c0 + 2, _NCHUNK - 1)
        pltpu.async_copy(xtraj.at[col_v.at[c2]], rows_a, sem_a)
        _process(c0 + 1, rows_b)
        return carry
    lax.fori_loop(0, _NCHUNK // 2, _pair, 0)
    pltpu.make_async_copy(
        xtraj.at[col_v.at[_NCHUNK - 1]], rows_a, sem_a).wait()

    plsc.subcore_barrier()

    pltpu.sync_copy(agg_sh.at[pl.ds(s * _RPT, _RPT)],
                    agg_out.at[c, pl.ds(s * _RPT, _RPT)])
    pltpu.sync_copy(ss_sh.at[pl.ds(s * _RPT, _RPT)],
                    ss_out.at[c, pl.ds(s * _RPT, _RPT)])


_sc_edge_call = pl.kernel(
    _sc_edge_body,
    out_type=(
        jax.ShapeDtypeStruct((_NC, _SEG_PAD, _D), jnp.float32),
        jax.ShapeDtypeStruct((_NC, _SEG_PAD), jnp.float32),
    ),
    mesh=plsc.VectorSubcoreMesh(core_axis_name="c", subcore_axis_name="s",
                                num_cores=_NC, num_subcores=_NS),
    scratch_types=[
        pltpu.VMEM_SHARED((_SEG_PAD, _D), jnp.float32),
        pltpu.VMEM_SHARED((_SEG_PAD,), jnp.float32),
        pltpu.VMEM((_NCHUNK, _CHUNK), jnp.int32),
        pltpu.VMEM((_NCHUNK, _CHUNK), jnp.int32),
        pltpu.VMEM((_NCHUNK, _CHUNK), jnp.float32),
        pltpu.VMEM((_CHUNK, _D), jnp.float32),
        pltpu.VMEM((_CHUNK, _D), jnp.float32),
        pltpu.VMEM((_CHUNK,), jnp.float32),
        pltpu.VMEM((_RPT,), jnp.float32),
        pltpu.SemaphoreType.DMA,
        pltpu.SemaphoreType.DMA,
    ],
)


_BM = 1000
_BN = 512
_NPOI_PAD = 5120


def _head_body(xci_ref, fuse_ref, agg0_ref, agg1_ref, ss0_ref, ss1_ref,
               g_ref, b_ref, w_ref, bias_ref, out_ref, h_scr):
    @pl.when(pl.program_id(1) == 0)
    def _():
        denom = ss0_ref[...] + ss1_ref[...] + 1e-30
        h = xci_ref[...] + (agg0_ref[...] + agg1_ref[...]) / denom
        mu = jnp.mean(h, axis=1, keepdims=True)
        var = jnp.mean((h - mu) * (h - mu), axis=1, keepdims=True)
        h = (h - mu) * jax.lax.rsqrt(var + 1e-5) * g_ref[...] + b_ref[...]
        h_scr[...] = h + fuse_ref[...]

    out_ref[...] = jnp.dot(h_scr[...], w_ref[...],
                           preferred_element_type=jnp.float32) + bias_ref[...]


_head_call = pl.pallas_call(
    _head_body,
    grid=(_NUM_CHECKIN // _BM, _NPOI_PAD // _BN),
    in_specs=[
        pl.BlockSpec((_BM, _D), lambda m, n: (m, 0)),
        pl.BlockSpec((_BM, _D), lambda m, n: (m, 0)),
        pl.BlockSpec((_BM, _D), lambda m, n: (m, 0)),
        pl.BlockSpec((_BM, _D), lambda m, n: (m, 0)),
        pl.BlockSpec((_BM, 1), lambda m, n: (m, 0)),
        pl.BlockSpec((_BM, 1), lambda m, n: (m, 0)),
        pl.BlockSpec((1, _D), lambda m, n: (0, 0)),
        pl.BlockSpec((1, _D), lambda m, n: (0, 0)),
        pl.BlockSpec((_D, _BN), lambda m, n: (0, n)),
        pl.BlockSpec((1, _BN), lambda m, n: (0, n)),
    ],
    out_specs=pl.BlockSpec((_BM, _BN), lambda m, n: (m, n)),
    out_shape=jax.ShapeDtypeStruct((_NUM_CHECKIN, _NUM_POI), jnp.float32),
    scratch_shapes=[pltpu.VMEM((_BM, _D), jnp.float32)],
)


def kernel(checkin_feature, delta_ts, delta_ss, ci2traj_attr, traj2traj_attr,
           time_w, time_b, dist_w, dist_b, tw, sw, ln_g, ln_b, W_out, b_out,
           ci2traj_row, ci2traj_col, traj2traj_row, traj2traj_col):
    del traj2traj_attr, traj2traj_row, traj2traj_col

    dts = delta_ts.reshape(_N, 1)
    dss = delta_ss.reshape(_N, 1)
    xtraj0, xci0, fuse_ci = _fusion_call(
        dts, dss, checkin_feature,
        time_w.reshape(1, _D), time_b.reshape(1, _D),
        dist_w.reshape(1, _D), dist_b.reshape(1, _D),
        tw.reshape(1, _D), sw.reshape(1, _D))

    pad = _E_PAD - _E1
    row_p = jnp.pad(ci2traj_row.astype(jnp.int32), (0, pad)).reshape(
        _NW * _NCHUNK, _CHUNK)
    col_p = jnp.pad(ci2traj_col.astype(jnp.int32), (0, pad)).reshape(
        _NW * _NCHUNK, _CHUNK)
    attr_p = jnp.pad(ci2traj_attr, (0, pad), constant_values=-1e30).reshape(
        _NW * _NCHUNK, _CHUNK)

    agg_pair, ss_pair = _sc_edge_call(xtraj0, row_p, col_p, attr_p)

    wp = jnp.pad(W_out, ((0, 0), (0, _NPOI_PAD - _NUM_POI)))
    bp = jnp.pad(b_out, (0, _NPOI_PAD - _NUM_POI)).reshape(1, _NPOI_PAD)

    return _head_call(
        xci0, fuse_ci,
        agg_pair[0, :_NUM_CHECKIN], agg_pair[1, :_NUM_CHECKIN],
        ss_pair[0, :_NUM_CHECKIN, None], ss_pair[1, :_NUM_CHECKIN, None],
        ln_g.reshape(1, _D), ln_b.reshape(1, _D), wp, bp)


# speedup vs baseline: 13.1701x; 1.5082x over previous
import functools

import jax
import jax.numpy as jnp
from jax import lax
from jax.experimental import pallas as pl
from jax.experimental.pallas import tpu as pltpu
from jax.experimental.pallas import tpu_sc as plsc

_NUM_TRAJ = 2000
_NUM_CHECKIN = 8000
_N = _NUM_TRAJ + _NUM_CHECKIN
_D = 128
_E1 = 160000
_NUM_POI = 5000

_NC = 2
_NS = 16
_NW = _NC * _NS
_CHUNK = 128
_NCHUNK = 40
_EPT = _CHUNK * _NCHUNK
_E_PAD = _EPT * _NW
_SEG_PAD = 8192
_RPT = _SEG_PAD // _NS


def _fusion_body(dts_ref, dss_ref, ci_ref, timew_ref, timeb_ref, distw_ref,
                 distb_ref, tw_ref, sw_ref, xtraj_ref, xci_ref, fuse_ref):
    t_emb = jnp.cos(dts_ref[...] * (1.0 / 3600.0) * timew_ref[...]
                    + timeb_ref[...])
    s_emb = jnp.maximum(dss_ref[...] * distw_ref[...] + distb_ref[...], 0.0)
    fuse = tw_ref[...] * t_emb + sw_ref[...] * s_emb
    xtraj_ref[...] = jnp.maximum(fuse[:_NUM_TRAJ], 0.0)
    xci_ref[...] = jnp.maximum(ci_ref[...] + fuse[_NUM_TRAJ:], 0.0)
    fuse_ref[...] = fuse[_NUM_TRAJ:]


_fusion_call = pl.pallas_call(
    _fusion_body,
    out_shape=(
        jax.ShapeDtypeStruct((_NUM_TRAJ, _D), jnp.float32),
        jax.ShapeDtypeStruct((_NUM_CHECKIN, _D), jnp.float32),
        jax.ShapeDtypeStruct((_NUM_CHECKIN, _D), jnp.float32),
    ),
)


def _sc_edge_body(xtraj, rowr, colr, attrr, agg_out, ss_out,
                  agg_sh, ss_sh, tab_sh, row_v, col_v, attr_v, rows_a, rows_b,
                  exp_v, zrow_v, sem_a, sem_b):
    c = lax.axis_index("c")
    s = lax.axis_index("s")
    wid = c * _NS + s

    zero16 = jnp.zeros((16,), jnp.float32)

    def _zrow(i, carry):
        for k in range(8):
            rows_a[i, pl.ds(k * 16, 16)] = zero16
        return carry
    lax.fori_loop(0, 128, _zrow, 0)
    for k in range(32):
        zrow_v[pl.ds(k * 16, 16)] = zero16
    for j in range(_RPT // 128):
        pltpu.sync_copy(rows_a, agg_sh.at[pl.ds(s * _RPT + j * 128, 128)])
    pltpu.sync_copy(zrow_v, ss_sh.at[pl.ds(s * _RPT, _RPT)])

    pltpu.sync_copy(rowr.at[pl.ds(wid * _NCHUNK, _NCHUNK)], row_v)
    pltpu.sync_copy(colr.at[pl.ds(wid * _NCHUNK, _NCHUNK)], col_v)
    pltpu.sync_copy(attrr.at[pl.ds(wid * _NCHUNK, _NCHUNK)], attr_v)

    @pl.when(s < _NS - 1)
    def _stage_full():
        off = pl.multiple_of(s * 128, 8)
        pltpu.sync_copy(xtraj.at[pl.ds(off, 128)], rows_b)
        pltpu.sync_copy(rows_b, tab_sh.at[pl.ds(off, 128)])

    @pl.when(s == _NS - 1)
    def _stage_tail():
        tail = _NUM_TRAJ - 128 * (_NS - 1)
        off = pl.multiple_of((_NS - 1) * 128, 8)
        pltpu.sync_copy(xtraj.at[pl.ds(off, tail)], rows_b.at[pl.ds(0, tail)])
        pltpu.sync_copy(rows_b.at[pl.ds(0, tail)], tab_sh.at[pl.ds(off, tail)])

    plsc.subcore_barrier()

    def _process(j, buf):
        def _group(g, carry2):
            a = attr_v[j, pl.ds(g * 16, 16)]
            w = jnp.exp(a)
            exp_v[pl.ds(g * 16, 16)] = w
            for t in range(16):
                ws = jnp.full((16,), w[t])
                e = g * 16 + t
                for k in range(8):
                    sl = pl.ds(k * 16, 16)
                    buf[e, sl] = buf[e, sl] * ws
            return carry2
        lax.fori_loop(0, 8, _group, 0)

        pltpu.sync_copy(buf, agg_sh.at[row_v.at[j]], add=True)
        pltpu.sync_copy(exp_v, ss_sh.at[row_v.at[j]], add=True)

    pltpu.async_copy(tab_sh.at[col_v.at[0]], rows_a, sem_a)

    def _pair(p, carry):
        c0 = 2 * p
        pltpu.make_async_copy(tab_sh.at[col_v.at[c0]], rows_a, sem_a).wait()
        pltpu.async_copy(tab_sh.at[col_v.at[c0 + 1]], rows_b, sem_b)
        _process(c0, rows_a)
        pltpu.make_async_copy(tab_sh.at[col_v.at[c0 + 1]], rows_b, sem_b).wait()
        c2 = jnp.minimum(c0 + 2, _NCHUNK - 1)
        pltpu.async_copy(tab_sh.at[col_v.at[c2]], rows_a, sem_a)
        _process(c0 + 1, rows_b)
        return carry
    lax.fori_loop(0, _NCHUNK // 2, _pair, 0)
    pltpu.make_async_copy(
        tab_sh.at[col_v.at[_NCHUNK - 1]], rows_a, sem_a).wait()

    plsc.subcore_barrier()

    pltpu.sync_copy(agg_sh.at[pl.ds(s * _RPT, _RPT)],
                    agg_out.at[c, pl.ds(s * _RPT, _RPT)])
    pltpu.sync_copy(ss_sh.at[pl.ds(s * _RPT, _RPT)],
                    ss_out.at[c, pl.ds(s * _RPT, _RPT)])


_sc_edge_call = pl.kernel(
    _sc_edge_body,
    out_type=(
        jax.ShapeDtypeStruct((_NC, _SEG_PAD, _D), jnp.float32),
        jax.ShapeDtypeStruct((_NC, _SEG_PAD), jnp.float32),
    ),
    mesh=plsc.VectorSubcoreMesh(core_axis_name="c", subcore_axis_name="s",
                                num_cores=_NC, num_subcores=_NS),
    scratch_types=[
        pltpu.VMEM_SHARED((_SEG_PAD, _D), jnp.float32),
        pltpu.VMEM_SHARED((_SEG_PAD,), jnp.float32),
        pltpu.VMEM_SHARED((_NUM_TRAJ, _D), jnp.float32),
        pltpu.VMEM((_NCHUNK, _CHUNK), jnp.int32),
        pltpu.VMEM((_NCHUNK, _CHUNK), jnp.int32),
        pltpu.VMEM((_NCHUNK, _CHUNK), jnp.float32),
        pltpu.VMEM((_CHUNK, _D), jnp.float32),
        pltpu.VMEM((_CHUNK, _D), jnp.float32),
        pltpu.VMEM((_CHUNK,), jnp.float32),
        pltpu.VMEM((_RPT,), jnp.float32),
        pltpu.SemaphoreType.DMA,
        pltpu.SemaphoreType.DMA,
    ],
)


_BM = 1000
_BN = 512
_NPOI_PAD = 5120


def _head_body(xci_ref, fuse_ref, agg0_ref, agg1_ref, ss0_ref, ss1_ref,
               g_ref, b_ref, w_ref, bias_ref, out_ref, h_scr):
    @pl.when(pl.program_id(1) == 0)
    def _():
        denom = ss0_ref[...] + ss1_ref[...] + 1e-30
        h = xci_ref[...] + (agg0_ref[...] + agg1_ref[...]) / denom
        mu = jnp.mean(h, axis=1, keepdims=True)
        var = jnp.mean((h - mu) * (h - mu), axis=1, keepdims=True)
        h = (h - mu) * jax.lax.rsqrt(var + 1e-5) * g_ref[...] + b_ref[...]
        h_scr[...] = h + fuse_ref[...]

    out_ref[...] = jnp.dot(h_scr[...], w_ref[...],
                           preferred_element_type=jnp.float32) + bias_ref[...]


_head_call = pl.pallas_call(
    _head_body,
    grid=(_NUM_CHECKIN // _BM, _NPOI_PAD // _BN),
    in_specs=[
        pl.BlockSpec((_BM, _D), lambda m, n: (m, 0)),
        pl.BlockSpec((_BM, _D), lambda m, n: (m, 0)),
        pl.BlockSpec((_BM, _D), lambda m, n: (m, 0)),
        pl.BlockSpec((_BM, _D), lambda m, n: (m, 0)),
        pl.BlockSpec((_BM, 1), lambda m, n: (m, 0)),
        pl.BlockSpec((_BM, 1), lambda m, n: (m, 0)),
        pl.BlockSpec((1, _D), lambda m, n: (0, 0)),
        pl.BlockSpec((1, _D), lambda m, n: (0, 0)),
        pl.BlockSpec((_D, _BN), lambda m, n: (0, n)),
        pl.BlockSpec((1, _BN), lambda m, n: (0, n)),
    ],
    out_specs=pl.BlockSpec((_BM, _BN), lambda m, n: (m, n)),
    out_shape=jax.ShapeDtypeStruct((_NUM_CHECKIN, _NUM_POI), jnp.float32),
    scratch_shapes=[pltpu.VMEM((_BM, _D), jnp.float32)],
)


def kernel(checkin_feature, delta_ts, delta_ss, ci2traj_attr, traj2traj_attr,
           time_w, time_b, dist_w, dist_b, tw, sw, ln_g, ln_b, W_out, b_out,
           ci2traj_row, ci2traj_col, traj2traj_row, traj2traj_col):
    del traj2traj_attr, traj2traj_row, traj2traj_col

    dts = delta_ts.reshape(_N, 1)
    dss = delta_ss.reshape(_N, 1)
    xtraj0, xci0, fuse_ci = _fusion_call(
        dts, dss, checkin_feature,
        time_w.reshape(1, _D), time_b.reshape(1, _D),
        dist_w.reshape(1, _D), dist_b.reshape(1, _D),
        tw.reshape(1, _D), sw.reshape(1, _D))

    pad = _E_PAD - _E1
    row_p = jnp.pad(ci2traj_row.astype(jnp.int32), (0, pad)).reshape(
        _NW * _NCHUNK, _CHUNK)
    col_p = jnp.pad(ci2traj_col.astype(jnp.int32), (0, pad)).reshape(
        _NW * _NCHUNK, _CHUNK)
    attr_p = jnp.pad(ci2traj_attr, (0, pad), constant_values=-1e30).reshape(
        _NW * _NCHUNK, _CHUNK)

    agg_pair, ss_pair = _sc_edge_call(xtraj0, row_p, col_p, attr_p)

    wp = jnp.pad(W_out, ((0, 0), (0, _NPOI_PAD - _NUM_POI)))
    bp = jnp.pad(b_out, (0, _NPOI_PAD - _NUM_POI)).reshape(1, _NPOI_PAD)

    return _head_call(
        xci0, fuse_ci,
        agg_pair[0, :_NUM_CHECKIN], agg_pair[1, :_NUM_CHECKIN],
        ss_pair[0, :_NUM_CHECKIN, None], ss_pair[1, :_NUM_CHECKIN, None],
        ln_g.reshape(1, _D), ln_b.reshape(1, _D), wp, bp)


# speedup vs baseline: 18.9609x; 1.4397x over previous
import functools

import jax
import jax.numpy as jnp
from jax import lax
from jax.experimental import pallas as pl
from jax.experimental.pallas import tpu as pltpu
from jax.experimental.pallas import tpu_sc as plsc

_NUM_TRAJ = 2000
_NUM_CHECKIN = 8000
_N = _NUM_TRAJ + _NUM_CHECKIN
_D = 128
_E1 = 160000
_NUM_POI = 5000

_NC = 2
_NS = 16
_NW = _NC * _NS
_CHUNK = 128
_NCHUNK = 40
_EPT = _CHUNK * _NCHUNK
_E_PAD = _EPT * _NW
_SEG_PAD = 8192
_RPT = _SEG_PAD // _NS


def _fusion_body(dts_ref, dss_ref, ci_ref, timew_ref, timeb_ref, distw_ref,
                 distb_ref, tw_ref, sw_ref, xtraj_ref, xci_ref, fuse_ref):
    t_emb = jnp.cos(dts_ref[...] * (1.0 / 3600.0) * timew_ref[...]
                    + timeb_ref[...])
    s_emb = jnp.maximum(dss_ref[...] * distw_ref[...] + distb_ref[...], 0.0)
    fuse = tw_ref[...] * t_emb + sw_ref[...] * s_emb
    xtraj_ref[...] = jnp.maximum(fuse[:_NUM_TRAJ], 0.0)
    xci_ref[...] = jnp.maximum(ci_ref[...] + fuse[_NUM_TRAJ:], 0.0)
    fuse_ref[...] = fuse[_NUM_TRAJ:]


_fusion_call = pl.pallas_call(
    _fusion_body,
    out_shape=(
        jax.ShapeDtypeStruct((_NUM_TRAJ, _D), jnp.float32),
        jax.ShapeDtypeStruct((_NUM_CHECKIN, _D), jnp.float32),
        jax.ShapeDtypeStruct((_NUM_CHECKIN, _D), jnp.float32),
    ),
)


def _sc_edge_body(xtraj, rowr, colr, attrr, agg_out, ss_out,
                  agg_sh, ss_sh, tab_sh, row_v, col_v, attr_v, rows_a, rows_b,
                  exp_a, exp_b, zrow_v, sem_a, sem_b, sem_sa, sem_sb,
                  sem_ea, sem_eb):
    c = lax.axis_index("c")
    s = lax.axis_index("s")
    wid = c * _NS + s

    zero16 = jnp.zeros((16,), jnp.float32)

    def _zrow(i, carry):
        for k in range(8):
            rows_a[i, pl.ds(k * 16, 16)] = zero16
        return carry
    lax.fori_loop(0, 128, _zrow, 0)
    for k in range(32):
        zrow_v[pl.ds(k * 16, 16)] = zero16
    for j in range(_RPT // 128):
        pltpu.sync_copy(rows_a, agg_sh.at[pl.ds(s * _RPT + j * 128, 128)])
    pltpu.sync_copy(zrow_v, ss_sh.at[pl.ds(s * _RPT, _RPT)])

    pltpu.sync_copy(rowr.at[pl.ds(wid * _NCHUNK, _NCHUNK)], row_v)
    pltpu.sync_copy(colr.at[pl.ds(wid * _NCHUNK, _NCHUNK)], col_v)
    pltpu.sync_copy(attrr.at[pl.ds(wid * _NCHUNK, _NCHUNK)], attr_v)

    @pl.when(s < _NS - 1)
    def _stage_full():
        off = pl.multiple_of(s * 128, 8)
        pltpu.sync_copy(xtraj.at[pl.ds(off, 128)], rows_b)
        pltpu.sync_copy(rows_b, tab_sh.at[pl.ds(off, 128)])

    @pl.when(s == _NS - 1)
    def _stage_tail():
        tail = _NUM_TRAJ - 128 * (_NS - 1)
        off = pl.multiple_of((_NS - 1) * 128, 8)
        pltpu.sync_copy(xtraj.at[pl.ds(off, tail)], rows_b.at[pl.ds(0, tail)])
        pltpu.sync_copy(rows_b.at[pl.ds(0, tail)], tab_sh.at[pl.ds(off, tail)])

    plsc.subcore_barrier()

    def _scale(j, buf, ebuf):
        def _group(g, carry2):
            a = attr_v[j, pl.ds(g * 16, 16)]
            w = jnp.exp(a)
            ebuf[pl.ds(g * 16, 16)] = w
            for t in range(16):
                ws = jnp.full((16,), w[t])
                e = g * 16 + t
                for k in range(8):
                    sl = pl.ds(k * 16, 16)
                    buf[e, sl] = buf[e, sl] * ws
            return carry2
        lax.fori_loop(0, 8, _group, 0)

    pltpu.async_copy(tab_sh.at[col_v.at[0]], rows_a, sem_a)

    def _pair(p, carry):
        c0 = 2 * p
        pltpu.make_async_copy(tab_sh.at[col_v.at[c0]], rows_a, sem_a).wait()
        pltpu.async_copy(tab_sh.at[col_v.at[c0 + 1]], rows_b, sem_b)
        _scale(c0, rows_a, exp_a)
        pltpu.async_copy(rows_a, agg_sh.at[row_v.at[c0]], sem_sa, add=True)
        pltpu.async_copy(exp_a, ss_sh.at[row_v.at[c0]], sem_ea, add=True)
        pltpu.make_async_copy(tab_sh.at[col_v.at[c0 + 1]], rows_b, sem_b).wait()
        _scale(c0 + 1, rows_b, exp_b)
        pltpu.async_copy(rows_b, agg_sh.at[row_v.at[c0 + 1]], sem_sb, add=True)
        pltpu.async_copy(exp_b, ss_sh.at[row_v.at[c0 + 1]], sem_eb, add=True)
        pltpu.make_async_copy(rows_a, agg_sh.at[row_v.at[c0]], sem_sa).wait()
        c2 = jnp.minimum(c0 + 2, _NCHUNK - 1)
        pltpu.async_copy(tab_sh.at[col_v.at[c2]], rows_a, sem_a)
        pltpu.make_async_copy(rows_b, agg_sh.at[row_v.at[c0 + 1]], sem_sb).wait()
        pltpu.make_async_copy(exp_a, ss_sh.at[row_v.at[c0]], sem_ea).wait()
        pltpu.make_async_copy(exp_b, ss_sh.at[row_v.at[c0 + 1]], sem_eb).wait()
        return carry
    lax.fori_loop(0, _NCHUNK // 2, _pair, 0)
    pltpu.make_async_copy(
        tab_sh.at[col_v.at[_NCHUNK - 1]], rows_a, sem_a).wait()

    plsc.subcore_barrier()

    pltpu.sync_copy(agg_sh.at[pl.ds(s * _RPT, _RPT)],
                    agg_out.at[c, pl.ds(s * _RPT, _RPT)])
    pltpu.sync_copy(ss_sh.at[pl.ds(s * _RPT, _RPT)],
                    ss_out.at[c, pl.ds(s * _RPT, _RPT)])


_sc_edge_call = pl.kernel(
    _sc_edge_body,
    out_type=(
        jax.ShapeDtypeStruct((_NC, _SEG_PAD, _D), jnp.float32),
        jax.ShapeDtypeStruct((_NC, _SEG_PAD), jnp.float32),
    ),
    mesh=plsc.VectorSubcoreMesh(core_axis_name="c", subcore_axis_name="s",
                                num_cores=_NC, num_subcores=_NS),
    scratch_types=[
        pltpu.VMEM_SHARED((_SEG_PAD, _D), jnp.float32),
        pltpu.VMEM_SHARED((_SEG_PAD,), jnp.float32),
        pltpu.VMEM_SHARED((_NUM_TRAJ, _D), jnp.float32),
        pltpu.VMEM((_NCHUNK, _CHUNK), jnp.int32),
        pltpu.VMEM((_NCHUNK, _CHUNK), jnp.int32),
        pltpu.VMEM((_NCHUNK, _CHUNK), jnp.float32),
        pltpu.VMEM((_CHUNK, _D), jnp.float32),
        pltpu.VMEM((_CHUNK, _D), jnp.float32),
        pltpu.VMEM((_CHUNK,), jnp.float32),
        pltpu.VMEM((_CHUNK,), jnp.float32),
        pltpu.VMEM((_RPT,), jnp.float32),
        pltpu.SemaphoreType.DMA,
        pltpu.SemaphoreType.DMA,
        pltpu.SemaphoreType.DMA,
        pltpu.SemaphoreType.DMA,
        pltpu.SemaphoreType.DMA,
        pltpu.SemaphoreType.DMA,
    ],
)


_BM = 1024
_BN = 512


def _head_body(xci_ref, fuse_ref, agg0_ref, agg1_ref, ss0_ref, ss1_ref,
               g_ref, b_ref, wt_ref, bias_ref, out_ref, ht_scr):
    @pl.when(pl.program_id(1) == 0)
    def _():
        denom = ss0_ref[...] + ss1_ref[...] + 1e-30
        h = xci_ref[...] + (agg0_ref[...] + agg1_ref[...]) / denom
        mu = jnp.mean(h, axis=1, keepdims=True)
        var = jnp.mean((h - mu) * (h - mu), axis=1, keepdims=True)
        h = (h - mu) * jax.lax.rsqrt(var + 1e-5) * g_ref[...] + b_ref[...]
        ht_scr[...] = (h + fuse_ref[...]).T

    out_ref[...] = jnp.dot(wt_ref[...], ht_scr[...],
                           preferred_element_type=jnp.float32) + bias_ref[...]


_head_call = pl.pallas_call(
    _head_body,
    grid=((_NUM_CHECKIN + _BM - 1) // _BM, (_NUM_POI + _BN - 1) // _BN),
    in_specs=[
        pl.BlockSpec((_BM, _D), lambda m, n: (m, 0)),
        pl.BlockSpec((_BM, _D), lambda m, n: (m, 0)),
        pl.BlockSpec((_BM, _D), lambda m, n: (m, 0)),
        pl.BlockSpec((_BM, _D), lambda m, n: (m, 0)),
        pl.BlockSpec((_BM, 1), lambda m, n: (m, 0)),
        pl.BlockSpec((_BM, 1), lambda m, n: (m, 0)),
        pl.BlockSpec((1, _D), lambda m, n: (0, 0)),
        pl.BlockSpec((1, _D), lambda m, n: (0, 0)),
        pl.BlockSpec((_BN, _D), lambda m, n: (n, 0)),
        pl.BlockSpec((_BN, 1), lambda m, n: (n, 0)),
    ],
    out_specs=pl.BlockSpec((_BN, _BM), lambda m, n: (n, m)),
    out_shape=jax.ShapeDtypeStruct((_NUM_POI, _NUM_CHECKIN), jnp.float32),
    scratch_shapes=[pltpu.VMEM((_D, _BM), jnp.float32)],
)


def kernel(checkin_feature, delta_ts, delta_ss, ci2traj_attr, traj2traj_attr,
           time_w, time_b, dist_w, dist_b, tw, sw, ln_g, ln_b, W_out, b_out,
           ci2traj_row, ci2traj_col, traj2traj_row, traj2traj_col):
    del traj2traj_attr, traj2traj_row, traj2traj_col

    dts = delta_ts.reshape(_N, 1)
    dss = delta_ss.reshape(_N, 1)
    xtraj0, xci0, fuse_ci = _fusion_call(
        dts, dss, checkin_feature,
        time_w.reshape(1, _D), time_b.reshape(1, _D),
        dist_w.reshape(1, _D), dist_b.reshape(1, _D),
        tw.reshape(1, _D), sw.reshape(1, _D))

    pad = _E_PAD - _E1
    row_p = jnp.pad(ci2traj_row.astype(jnp.int32), (0, pad)).reshape(
        _NW * _NCHUNK, _CHUNK)
    col_p = jnp.pad(ci2traj_col.astype(jnp.int32), (0, pad)).reshape(
        _NW * _NCHUNK, _CHUNK)
    attr_p = jnp.pad(ci2traj_attr, (0, pad), constant_values=-1e30).reshape(
        _NW * _NCHUNK, _CHUNK)

    agg_pair, ss_pair = _sc_edge_call(xtraj0, row_p, col_p, attr_p)

    out_t = _head_call(
        xci0, fuse_ci,
        agg_pair[0, :_NUM_CHECKIN], agg_pair[1, :_NUM_CHECKIN],
        ss_pair[0, :_NUM_CHECKIN, None], ss_pair[1, :_NUM_CHECKIN, None],
        ln_g.reshape(1, _D), ln_b.reshape(1, _D),
        W_out.T, b_out.reshape(_NUM_POI, 1))
    return out_t.T


# speedup vs baseline: 19.2041x; 1.0128x over previous
import functools

import jax
import jax.numpy as jnp
from jax import lax
from jax.experimental import pallas as pl
from jax.experimental.pallas import tpu as pltpu
from jax.experimental.pallas import tpu_sc as plsc

_NUM_TRAJ = 2000
_NUM_CHECKIN = 8000
_N = _NUM_TRAJ + _NUM_CHECKIN
_D = 128
_E1 = 160000
_NUM_POI = 5000

_NC = 2
_NS = 16
_NW = _NC * _NS
_CHUNK = 128
_NCHUNK = 40
_EPT = _CHUNK * _NCHUNK
_E_PAD = _EPT * _NW
_SEG_PAD = 8192
_RPT = _SEG_PAD // _NS


def _fusion_body(dts_ref, dss_ref, ci_ref, timew_ref, timeb_ref, distw_ref,
                 distb_ref, tw_ref, sw_ref, xtraj_ref, xci_ref, fuse_ref):
    t_emb = jnp.cos(dts_ref[...] * (1.0 / 3600.0) * timew_ref[...]
                    + timeb_ref[...])
    s_emb = jnp.maximum(dss_ref[...] * distw_ref[...] + distb_ref[...], 0.0)
    fuse = tw_ref[...] * t_emb + sw_ref[...] * s_emb
    xtraj_ref[...] = jnp.maximum(fuse[:_NUM_TRAJ], 0.0)
    xci_ref[...] = jnp.maximum(ci_ref[...] + fuse[_NUM_TRAJ:], 0.0)
    fuse_ref[...] = fuse[_NUM_TRAJ:]


_fusion_call = pl.pallas_call(
    _fusion_body,
    out_shape=(
        jax.ShapeDtypeStruct((_NUM_TRAJ, _D), jnp.float32),
        jax.ShapeDtypeStruct((_NUM_CHECKIN, _D), jnp.float32),
        jax.ShapeDtypeStruct((_NUM_CHECKIN, _D), jnp.float32),
    ),
)


def _sc_edge_body(xtraj, rowr, colr, attrr, agg_out, ss_out,
                  agg_sh, ss_sh, tab_sh, row_v, col_v, attr_v, rows_a, rows_b,
                  exp_a, exp_b, zrow_v, sem_a, sem_b, sem_sa, sem_sb,
                  sem_ea, sem_eb):
    c = lax.axis_index("c")
    s = lax.axis_index("s")
    wid = c * _NS + s

    zero16 = jnp.zeros((16,), jnp.float32)

    def _zrow(i, carry):
        for k in range(8):
            rows_a[i, pl.ds(k * 16, 16)] = zero16
        return carry
    lax.fori_loop(0, 128, _zrow, 0)
    for k in range(32):
        zrow_v[pl.ds(k * 16, 16)] = zero16
    for j in range(_RPT // 128):
        pltpu.sync_copy(rows_a, agg_sh.at[pl.ds(s * _RPT + j * 128, 128)])
    pltpu.sync_copy(zrow_v, ss_sh.at[pl.ds(s * _RPT, _RPT)])

    pltpu.sync_copy(rowr.at[pl.ds(wid * _NCHUNK, _NCHUNK)], row_v)
    pltpu.sync_copy(colr.at[pl.ds(wid * _NCHUNK, _NCHUNK)], col_v)
    pltpu.sync_copy(attrr.at[pl.ds(wid * _NCHUNK, _NCHUNK)], attr_v)

    @pl.when(s < _NS - 1)
    def _stage_full():
        off = pl.multiple_of(s * 128, 8)
        pltpu.sync_copy(xtraj.at[pl.ds(off, 128)], rows_b)
        pltpu.sync_copy(rows_b, tab_sh.at[pl.ds(off, 128)])

    @pl.when(s == _NS - 1)
    def _stage_tail():
        tail = _NUM_TRAJ - 128 * (_NS - 1)
        off = pl.multiple_of((_NS - 1) * 128, 8)
        pltpu.sync_copy(xtraj.at[pl.ds(off, tail)], rows_b.at[pl.ds(0, tail)])
        pltpu.sync_copy(rows_b.at[pl.ds(0, tail)], tab_sh.at[pl.ds(off, tail)])

    plsc.subcore_barrier()

    def _scale(j, buf, ebuf):
        def _group(g, carry2):
            a = attr_v[j, pl.ds(g * 16, 16)]
            w = jnp.exp(a)
            ebuf[pl.ds(g * 16, 16)] = w
            for t in range(16):
                ws = jnp.full((16,), w[t])
                e = g * 16 + t
                for k in range(8):
                    sl = pl.ds(k * 16, 16)
                    buf[e, sl] = buf[e, sl] * ws
            return carry2
        lax.fori_loop(0, 8, _group, 0)

    pltpu.async_copy(tab_sh.at[col_v.at[0]], rows_a, sem_a)

    def _pair(p, carry):
        c0 = 2 * p
        pltpu.make_async_copy(tab_sh.at[col_v.at[c0]], rows_a, sem_a).wait()
        pltpu.async_copy(tab_sh.at[col_v.at[c0 + 1]], rows_b, sem_b)
        _scale(c0, rows_a, exp_a)
        pltpu.async_copy(rows_a, agg_sh.at[row_v.at[c0]], sem_sa, add=True)
        pltpu.async_copy(exp_a, ss_sh.at[row_v.at[c0]], sem_ea, add=True)
        pltpu.make_async_copy(tab_sh.at[col_v.at[c0 + 1]], rows_b, sem_b).wait()
        _scale(c0 + 1, rows_b, exp_b)
        pltpu.async_copy(rows_b, agg_sh.at[row_v.at[c0 + 1]], sem_sb, add=True)
        pltpu.async_copy(exp_b, ss_sh.at[row_v.at[c0 + 1]], sem_eb, add=True)
        pltpu.make_async_copy(rows_a, agg_sh.at[row_v.at[c0]], sem_sa).wait()
        c2 = jnp.minimum(c0 + 2, _NCHUNK - 1)
        pltpu.async_copy(tab_sh.at[col_v.at[c2]], rows_a, sem_a)
        pltpu.make_async_copy(rows_b, agg_sh.at[row_v.at[c0 + 1]], sem_sb).wait()
        pltpu.make_async_copy(exp_a, ss_sh.at[row_v.at[c0]], sem_ea).wait()
        pltpu.make_async_copy(exp_b, ss_sh.at[row_v.at[c0 + 1]], sem_eb).wait()
        return carry
    lax.fori_loop(0, _NCHUNK // 2, _pair, 0)
    pltpu.make_async_copy(
        tab_sh.at[col_v.at[_NCHUNK - 1]], rows_a, sem_a).wait()

    plsc.subcore_barrier()

    pltpu.sync_copy(agg_sh.at[pl.ds(s * _RPT, _RPT)],
                    agg_out.at[c, pl.ds(s * _RPT, _RPT)])
    pltpu.sync_copy(ss_sh.at[pl.ds(s * _RPT, _RPT)],
                    ss_out.at[c, pl.ds(s * _RPT, _RPT)])


_sc_edge_call = pl.kernel(
    _sc_edge_body,
    out_type=(
        jax.ShapeDtypeStruct((_NC, _SEG_PAD, _D), jnp.float32),
        jax.ShapeDtypeStruct((_NC, _SEG_PAD), jnp.float32),
    ),
    mesh=plsc.VectorSubcoreMesh(core_axis_name="c", subcore_axis_name="s",
                                num_cores=_NC, num_subcores=_NS),
    scratch_types=[
        pltpu.VMEM_SHARED((_SEG_PAD, _D), jnp.float32),
        pltpu.VMEM_SHARED((_SEG_PAD,), jnp.float32),
        pltpu.VMEM_SHARED((_NUM_TRAJ, _D), jnp.float32),
        pltpu.VMEM((_NCHUNK, _CHUNK), jnp.int32),
        pltpu.VMEM((_NCHUNK, _CHUNK), jnp.int32),
        pltpu.VMEM((_NCHUNK, _CHUNK), jnp.float32),
        pltpu.VMEM((_CHUNK, _D), jnp.float32),
        pltpu.VMEM((_CHUNK, _D), jnp.float32),
        pltpu.VMEM((_CHUNK,), jnp.float32),
        pltpu.VMEM((_CHUNK,), jnp.float32),
        pltpu.VMEM((_RPT,), jnp.float32),
        pltpu.SemaphoreType.DMA,
        pltpu.SemaphoreType.DMA,
        pltpu.SemaphoreType.DMA,
        pltpu.SemaphoreType.DMA,
        pltpu.SemaphoreType.DMA,
        pltpu.SemaphoreType.DMA,
    ],
)


_BM = 1024
_BN = 512


def _head_body(xci_ref, fuse_ref, agg0_ref, agg1_ref, ss0_ref, ss1_ref,
               g_ref, b_ref, wt_ref, bias_ref, out_ref, ht_scr):
    @pl.when(pl.program_id(1) == 0)
    def _():
        denom = ss0_ref[...] + ss1_ref[...] + 1e-30
        h = xci_ref[...] + (agg0_ref[...] + agg1_ref[...]) / denom
        mu = jnp.mean(h, axis=1, keepdims=True)
        var = jnp.mean((h - mu) * (h - mu), axis=1, keepdims=True)
        h = (h - mu) * jax.lax.rsqrt(var + 1e-5) * g_ref[...] + b_ref[...]
        ht_scr[...] = (h + fuse_ref[...]).T.astype(jnp.bfloat16)

    out_ref[...] = jnp.dot(wt_ref[...], ht_scr[...],
                           preferred_element_type=jnp.float32) + bias_ref[...]


_head_call = pl.pallas_call(
    _head_body,
    grid=((_NUM_CHECKIN + _BM - 1) // _BM, (_NUM_POI + _BN - 1) // _BN),
    in_specs=[
        pl.BlockSpec((_BM, _D), lambda m, n: (m, 0)),
        pl.BlockSpec((_BM, _D), lambda m, n: (m, 0)),
        pl.BlockSpec((_BM, _D), lambda m, n: (m, 0)),
        pl.BlockSpec((_BM, _D), lambda m, n: (m, 0)),
        pl.BlockSpec((_BM, 1), lambda m, n: (m, 0)),
        pl.BlockSpec((_BM, 1), lambda m, n: (m, 0)),
        pl.BlockSpec((1, _D), lambda m, n: (0, 0)),
        pl.BlockSpec((1, _D), lambda m, n: (0, 0)),
        pl.BlockSpec((_BN, _D), lambda m, n: (n, 0)),
        pl.BlockSpec((_BN, 1), lambda m, n: (n, 0)),
    ],
    out_specs=pl.BlockSpec((_BN, _BM), lambda m, n: (n, m)),
    out_shape=jax.ShapeDtypeStruct((_NUM_POI, _NUM_CHECKIN), jnp.float32),
    scratch_shapes=[pltpu.VMEM((_D, _BM), jnp.bfloat16)],
)


def kernel(checkin_feature, delta_ts, delta_ss, ci2traj_attr, traj2traj_attr,
           time_w, time_b, dist_w, dist_b, tw, sw, ln_g, ln_b, W_out, b_out,
           ci2traj_row, ci2traj_col, traj2traj_row, traj2traj_col):
    del traj2traj_attr, traj2traj_row, traj2traj_col

    dts = delta_ts.reshape(_N, 1)
    dss = delta_ss.reshape(_N, 1)
    xtraj0, xci0, fuse_ci = _fusion_call(
        dts, dss, checkin_feature,
        time_w.reshape(1, _D), time_b.reshape(1, _D),
        dist_w.reshape(1, _D), dist_b.reshape(1, _D),
        tw.reshape(1, _D), sw.reshape(1, _D))

    pad = _E_PAD - _E1
    row_p = jnp.pad(ci2traj_row.astype(jnp.int32), (0, pad)).reshape(
        _NW * _NCHUNK, _CHUNK)
    col_p = jnp.pad(ci2traj_col.astype(jnp.int32), (0, pad)).reshape(
        _NW * _NCHUNK, _CHUNK)
    attr_p = jnp.pad(ci2traj_attr, (0, pad), constant_values=-1e30).reshape(
        _NW * _NCHUNK, _CHUNK)

    agg_pair, ss_pair = _sc_edge_call(xtraj0, row_p, col_p, attr_p)

    out_t = _head_call(
        xci0, fuse_ci,
        agg_pair[0, :_NUM_CHECKIN], agg_pair[1, :_NUM_CHECKIN],
        ss_pair[0, :_NUM_CHECKIN, None], ss_pair[1, :_NUM_CHECKIN, None],
        ln_g.reshape(1, _D), ln_b.reshape(1, _D),
        W_out.T.astype(jnp.bfloat16), b_out.reshape(_NUM_POI, 1))
    return out_t.T


# speedup vs baseline: 21.9647x; 1.1438x over previous
import functools

import jax
import jax.numpy as jnp
from jax import lax
from jax.experimental import pallas as pl
from jax.experimental.pallas import tpu as pltpu
from jax.experimental.pallas import tpu_sc as plsc

_NUM_TRAJ = 2000
_NUM_CHECKIN = 8000
_N = _NUM_TRAJ + _NUM_CHECKIN
_D = 128
_E1 = 160000
_NUM_POI = 5000

_NC = 2
_NS = 16
_NW = _NC * _NS
_CHUNK = 128
_NCHUNK = 40
_EPT = _CHUNK * _NCHUNK
_E_PAD = _EPT * _NW
_SEG_PAD = 8192
_RPT = _SEG_PAD // _NS


def _fast_cos(x):
    q = jnp.round(x * jnp.float32(0.6366197723675814))
    r = x - q * jnp.float32(1.57079601287841796875)
    r = r - q * jnp.float32(3.139164786504813e-7)
    r = r - q * jnp.float32(5.389759655482421e-15)
    r2 = r * r
    s = r + r * r2 * (jnp.float32(-1.6666654611e-1)
                      + r2 * (jnp.float32(8.3321608736e-3)
                              + r2 * jnp.float32(-1.9515295891e-4)))
    c = 1.0 - 0.5 * r2 + r2 * r2 * (jnp.float32(4.166664568298827e-2)
                                    + r2 * (jnp.float32(-1.388731625493765e-3)
                                            + r2 * jnp.float32(2.443315711809948e-5)))
    k = q.astype(jnp.int32)
    odd = (k & 1) == 1
    neg = ((k + (k & 1)) & 2) != 0
    base = jnp.where(odd, s, c)
    return jnp.where(neg, -base, base)


def _fusion_body(dts_ref, dss_ref, ci_ref, timew_ref, timeb_ref, distw_ref,
                 distb_ref, tw_ref, sw_ref, xtraj_ref, xci_ref, fuse_ref):
    t_emb = _fast_cos(dts_ref[...] * (1.0 / 3600.0) * timew_ref[...]
                      + timeb_ref[...])
    s_emb = jnp.maximum(dss_ref[...] * distw_ref[...] + distb_ref[...], 0.0)
    fuse = tw_ref[...] * t_emb + sw_ref[...] * s_emb
    xtraj_ref[...] = jnp.maximum(fuse[:_NUM_TRAJ], 0.0)
    xci_ref[...] = jnp.maximum(ci_ref[...] + fuse[_NUM_TRAJ:], 0.0)
    fuse_ref[...] = fuse[_NUM_TRAJ:]


_fusion_call = pl.pallas_call(
    _fusion_body,
    out_shape=(
        jax.ShapeDtypeStruct((_NUM_TRAJ, _D), jnp.float32),
        jax.ShapeDtypeStruct((_NUM_CHECKIN, _D), jnp.float32),
        jax.ShapeDtypeStruct((_NUM_CHECKIN, _D), jnp.float32),
    ),
)


def _sc_edge_body(xtraj, rowr, colr, attrr, agg_out, ss_out,
                  agg_sh, ss_sh, tab_sh, row_v, col_v, attr_v, rows_a, rows_b,
                  exp_a, exp_b, zrow_v, sem_a, sem_b, sem_sa, sem_sb,
                  sem_ea, sem_eb):
    c = lax.axis_index("c")
    s = lax.axis_index("s")
    wid = c * _NS + s

    zero16 = jnp.zeros((16,), jnp.float32)

    def _zrow(i, carry):
        for k in range(8):
            rows_a[i, pl.ds(k * 16, 16)] = zero16
        return carry
    lax.fori_loop(0, 128, _zrow, 0)
    for k in range(32):
        zrow_v[pl.ds(k * 16, 16)] = zero16
    for j in range(_RPT // 128):
        pltpu.sync_copy(rows_a, agg_sh.at[pl.ds(s * _RPT + j * 128, 128)])
    pltpu.sync_copy(zrow_v, ss_sh.at[pl.ds(s * _RPT, _RPT)])

    pltpu.sync_copy(rowr.at[pl.ds(wid * _NCHUNK, _NCHUNK)], row_v)
    pltpu.sync_copy(colr.at[pl.ds(wid * _NCHUNK, _NCHUNK)], col_v)
    pltpu.sync_copy(attrr.at[pl.ds(wid * _NCHUNK, _NCHUNK)], attr_v)

    @pl.when(s < _NS - 1)
    def _stage_full():
        off = pl.multiple_of(s * 128, 8)
        pltpu.sync_copy(xtraj.at[pl.ds(off, 128)], rows_b)
        pltpu.sync_copy(rows_b, tab_sh.at[pl.ds(off, 128)])

    @pl.when(s == _NS - 1)
    def _stage_tail():
        tail = _NUM_TRAJ - 128 * (_NS - 1)
        off = pl.multiple_of((_NS - 1) * 128, 8)
        pltpu.sync_copy(xtraj.at[pl.ds(off, tail)], rows_b.at[pl.ds(0, tail)])
        pltpu.sync_copy(rows_b.at[pl.ds(0, tail)], tab_sh.at[pl.ds(off, tail)])

    plsc.subcore_barrier()

    def _scale(j, buf, ebuf):
        def _group(g, carry2):
            a = attr_v[j, pl.ds(g * 16, 16)]
            w = jnp.exp(a)
            ebuf[pl.ds(g * 16, 16)] = w
            for t in range(16):
                ws = jnp.full((16,), w[t])
                e = g * 16 + t
                for k in range(8):
                    sl = pl.ds(k * 16, 16)
                    buf[e, sl] = buf[e, sl] * ws
            return carry2
        lax.fori_loop(0, 8, _group, 0)

    pltpu.async_copy(tab_sh.at[col_v.at[0]], rows_a, sem_a)

    def _pair(p, carry):
        c0 = 2 * p
        pltpu.make_async_copy(tab_sh.at[col_v.at[c0]], rows_a, sem_a).wait()
        pltpu.async_copy(tab_sh.at[col_v.at[c0 + 1]], rows_b, sem_b)
        _scale(c0, rows_a, exp_a)
        pltpu.async_copy(rows_a, agg_sh.at[row_v.at[c0]], sem_sa, add=True)
        pltpu.async_copy(exp_a, ss_sh.at[row_v.at[c0]], sem_ea, add=True)
        pltpu.make_async_copy(tab_sh.at[col_v.at[c0 + 1]], rows_b, sem_b).wait()
        _scale(c0 + 1, rows_b, exp_b)
        pltpu.async_copy(rows_b, agg_sh.at[row_v.at[c0 + 1]], sem_sb, add=True)
        pltpu.async_copy(exp_b, ss_sh.at[row_v.at[c0 + 1]], sem_eb, add=True)
        pltpu.make_async_copy(rows_a, agg_sh.at[row_v.at[c0]], sem_sa).wait()
        c2 = jnp.minimum(c0 + 2, _NCHUNK - 1)
        pltpu.async_copy(tab_sh.at[col_v.at[c2]], rows_a, sem_a)
        pltpu.make_async_copy(rows_b, agg_sh.at[row_v.at[c0 + 1]], sem_sb).wait()
        pltpu.make_async_copy(exp_a, ss_sh.at[row_v.at[c0]], sem_ea).wait()
        pltpu.make_async_copy(exp_b, ss_sh.at[row_v.at[c0 + 1]], sem_eb).wait()
        return carry
    lax.fori_loop(0, _NCHUNK // 2, _pair, 0)
    pltpu.make_async_copy(
        tab_sh.at[col_v.at[_NCHUNK - 1]], rows_a, sem_a).wait()

    plsc.subcore_barrier()

    pltpu.sync_copy(agg_sh.at[pl.ds(s * _RPT, _RPT)],
                    agg_out.at[c, pl.ds(s * _RPT, _RPT)])
    pltpu.sync_copy(ss_sh.at[pl.ds(s * _RPT, _RPT)],
                    ss_out.at[c, pl.ds(s * _RPT, _RPT)])


_sc_edge_call = pl.kernel(
    _sc_edge_body,
    out_type=(
        jax.ShapeDtypeStruct((_NC, _SEG_PAD, _D), jnp.float32),
        jax.ShapeDtypeStruct((_NC, _SEG_PAD), jnp.float32),
    ),
    mesh=plsc.VectorSubcoreMesh(core_axis_name="c", subcore_axis_name="s",
                                num_cores=_NC, num_subcores=_NS),
    scratch_types=[
        pltpu.VMEM_SHARED((_SEG_PAD, _D), jnp.float32),
        pltpu.VMEM_SHARED((_SEG_PAD,), jnp.float32),
        pltpu.VMEM_SHARED((_NUM_TRAJ, _D), jnp.float32),
        pltpu.VMEM((_NCHUNK, _CHUNK), jnp.int32),
        pltpu.VMEM((_NCHUNK, _CHUNK), jnp.int32),
        pltpu.VMEM((_NCHUNK, _CHUNK), jnp.float32),
        pltpu.VMEM((_CHUNK, _D), jnp.float32),
        pltpu.VMEM((_CHUNK, _D), jnp.float32),
        pltpu.VMEM((_CHUNK,), jnp.float32),
        pltpu.VMEM((_CHUNK,), jnp.float32),
        pltpu.VMEM((_RPT,), jnp.float32),
        pltpu.SemaphoreType.DMA,
        pltpu.SemaphoreType.DMA,
        pltpu.SemaphoreType.DMA,
        pltpu.SemaphoreType.DMA,
        pltpu.SemaphoreType.DMA,
        pltpu.SemaphoreType.DMA,
    ],
)


_BM = 1024
_BN = 1024


def _head_body(xci_ref, fuse_ref, agg0_ref, agg1_ref, ss0_ref, ss1_ref,
               g_ref, b_ref, wt_ref, bias_ref, out_ref, ht_scr):
    @pl.when(pl.program_id(1) == 0)
    def _():
        denom = ss0_ref[...] + ss1_ref[...] + 1e-30
        h = xci_ref[...] + (agg0_ref[...] + agg1_ref[...]) / denom
        mu = jnp.mean(h, axis=1, keepdims=True)
        var = jnp.mean((h - mu) * (h - mu), axis=1, keepdims=True)
        h = (h - mu) * jax.lax.rsqrt(var + 1e-5) * g_ref[...] + b_ref[...]
        ht_scr[...] = (h + fuse_ref[...]).T.astype(jnp.bfloat16)

    out_ref[...] = jnp.dot(wt_ref[...], ht_scr[...],
                           preferred_element_type=jnp.float32) + bias_ref[...]


_head_call = pl.pallas_call(
    _head_body,
    grid=((_NUM_CHECKIN + _BM - 1) // _BM, (_NUM_POI + _BN - 1) // _BN),
    in_specs=[
        pl.BlockSpec((_BM, _D), lambda m, n: (m, 0)),
        pl.BlockSpec((_BM, _D), lambda m, n: (m, 0)),
        pl.BlockSpec((_BM, _D), lambda m, n: (m, 0)),
        pl.BlockSpec((_BM, _D), lambda m, n: (m, 0)),
        pl.BlockSpec((_BM, 1), lambda m, n: (m, 0)),
        pl.BlockSpec((_BM, 1), lambda m, n: (m, 0)),
        pl.BlockSpec((1, _D), lambda m, n: (0, 0)),
        pl.BlockSpec((1, _D), lambda m, n: (0, 0)),
        pl.BlockSpec((_BN, _D), lambda m, n: (n, 0)),
        pl.BlockSpec((_BN, 1), lambda m, n: (n, 0)),
    ],
    out_specs=pl.BlockSpec((_BN, _BM), lambda m, n: (n, m)),
    out_shape=jax.ShapeDtypeStruct((_NUM_POI, _NUM_CHECKIN), jnp.float32),
    scratch_shapes=[pltpu.VMEM((_D, _BM), jnp.bfloat16)],
)


def kernel(checkin_feature, delta_ts, delta_ss, ci2traj_attr, traj2traj_attr,
           time_w, time_b, dist_w, dist_b, tw, sw, ln_g, ln_b, W_out, b_out,
           ci2traj_row, ci2traj_col, traj2traj_row, traj2traj_col):
    del traj2traj_attr, traj2traj_row, traj2traj_col

    dts = delta_ts.reshape(_N, 1)
    dss = delta_ss.reshape(_N, 1)
    xtraj0, xci0, fuse_ci = _fusion_call(
        dts, dss, checkin_feature,
        time_w.reshape(1, _D), time_b.reshape(1, _D),
        dist_w.reshape(1, _D), dist_b.reshape(1, _D),
        tw.reshape(1, _D), sw.reshape(1, _D))

    pad = _E_PAD - _E1
    row_p = jnp.pad(ci2traj_row.astype(jnp.int32), (0, pad)).reshape(
        _NW * _NCHUNK, _CHUNK)
    col_p = jnp.pad(ci2traj_col.astype(jnp.int32), (0, pad)).reshape(
        _NW * _NCHUNK, _CHUNK)
    attr_p = jnp.pad(ci2traj_attr, (0, pad), constant_values=-1e30).reshape(
        _NW * _NCHUNK, _CHUNK)

    agg_pair, ss_pair = _sc_edge_call(xtraj0, row_p, col_p, attr_p)

    out_t = _head_call(
        xci0, fuse_ci,
        agg_pair[0, :_NUM_CHECKIN], agg_pair[1, :_NUM_CHECKIN],
        ss_pair[0, :_NUM_CHECKIN, None], ss_pair[1, :_NUM_CHECKIN, None],
        ln_g.reshape(1, _D), ln_b.reshape(1, _D),
        W_out.T.astype(jnp.bfloat16), b_out.reshape(_NUM_POI, 1))
    return out_t.T


# speedup vs baseline: 23.6337x; 1.0760x over previous
import functools

import jax
import jax.numpy as jnp
from jax import lax
from jax.experimental import pallas as pl
from jax.experimental.pallas import tpu as pltpu
from jax.experimental.pallas import tpu_sc as plsc

_NUM_TRAJ = 2000
_NUM_CHECKIN = 8000
_N = _NUM_TRAJ + _NUM_CHECKIN
_D = 128
_E1 = 160000
_NUM_POI = 5000

_NC = 2
_NS = 16
_NW = _NC * _NS
_CHUNK = 128
_NCHUNK = 40
_EPT = _CHUNK * _NCHUNK
_E_PAD = _EPT * _NW
_SEG_PAD = 8192
_RPT = _SEG_PAD // _NS


def _fast_cos(x):
    q = jnp.round(x * jnp.float32(0.6366197723675814))
    r = x - q * jnp.float32(1.57079601287841796875)
    r = r - q * jnp.float32(3.139164786504813e-7)
    r = r - q * jnp.float32(5.389759655482421e-15)
    r2 = r * r
    s = r + r * r2 * (jnp.float32(-1.6666654611e-1)
                      + r2 * (jnp.float32(8.3321608736e-3)
                              + r2 * jnp.float32(-1.9515295891e-4)))
    c = 1.0 - 0.5 * r2 + r2 * r2 * (jnp.float32(4.166664568298827e-2)
                                    + r2 * (jnp.float32(-1.388731625493765e-3)
                                            + r2 * jnp.float32(2.443315711809948e-5)))
    k = q.astype(jnp.int32)
    odd = (k & 1) == 1
    neg = ((k + (k & 1)) & 2) != 0
    base = jnp.where(odd, s, c)
    return jnp.where(neg, -base, base)


def _fusion_traj_body(dts_ref, dss_ref, timew_ref, timeb_ref, distw_ref,
                      distb_ref, tw_ref, sw_ref, xtraj_ref):
    t_emb = _fast_cos(dts_ref[...] * (1.0 / 3600.0) * timew_ref[...]
                      + timeb_ref[...])
    s_emb = jnp.maximum(dss_ref[...] * distw_ref[...] + distb_ref[...], 0.0)
    fuse = tw_ref[...] * t_emb + sw_ref[...] * s_emb
    xtraj_ref[...] = jnp.maximum(fuse, 0.0)


_fusion_traj_call = pl.pallas_call(
    _fusion_traj_body,
    out_shape=jax.ShapeDtypeStruct((_NUM_TRAJ, _D), jnp.float32),
)


def _fusion_ci_body(dts_ref, dss_ref, ci_ref, timew_ref, timeb_ref, distw_ref,
                    distb_ref, tw_ref, sw_ref, xci_ref, fuse_ref):
    t_emb = _fast_cos(dts_ref[...] * (1.0 / 3600.0) * timew_ref[...]
                      + timeb_ref[...])
    s_emb = jnp.maximum(dss_ref[...] * distw_ref[...] + distb_ref[...], 0.0)
    fuse = tw_ref[...] * t_emb + sw_ref[...] * s_emb
    xci_ref[...] = jnp.maximum(ci_ref[...] + fuse, 0.0)
    fuse_ref[...] = fuse


_fusion_ci_call = pl.pallas_call(
    _fusion_ci_body,
    out_shape=(
        jax.ShapeDtypeStruct((_NUM_CHECKIN, _D), jnp.float32),
        jax.ShapeDtypeStruct((_NUM_CHECKIN, _D), jnp.float32),
    ),
)


def _sc_edge_body(xtraj, rowr, colr, attrr, agg_out, ss_out,
                  agg_sh, ss_sh, tab_sh, row_v, col_v, attr_v, rows_a, rows_b,
                  exp_a, exp_b, zrow_v, sem_a, sem_b, sem_sa, sem_sb,
                  sem_ea, sem_eb):
    c = lax.axis_index("c")
    s = lax.axis_index("s")
    wid = c * _NS + s

    zero16 = jnp.zeros((16,), jnp.float32)

    def _zrow(i, carry):
        for k in range(8):
            rows_a[i, pl.ds(k * 16, 16)] = zero16
        return carry
    lax.fori_loop(0, 128, _zrow, 0)
    for k in range(32):
        zrow_v[pl.ds(k * 16, 16)] = zero16
    for j in range(_RPT // 128):
        pltpu.sync_copy(rows_a, agg_sh.at[pl.ds(s * _RPT + j * 128, 128)])
    pltpu.sync_copy(zrow_v, ss_sh.at[pl.ds(s * _RPT, _RPT)])

    pltpu.sync_copy(rowr.at[pl.ds(wid * _NCHUNK, _NCHUNK)], row_v)
    pltpu.sync_copy(colr.at[pl.ds(wid * _NCHUNK, _NCHUNK)], col_v)
    pltpu.sync_copy(attrr.at[pl.ds(wid * _NCHUNK, _NCHUNK)], attr_v)

    @pl.when(s < _NS - 1)
    def _stage_full():
        off = pl.multiple_of(s * 128, 8)
        pltpu.sync_copy(xtraj.at[pl.ds(off, 128)], rows_b)
        pltpu.sync_copy(rows_b, tab_sh.at[pl.ds(off, 128)])

    @pl.when(s == _NS - 1)
    def _stage_tail():
        tail = _NUM_TRAJ - 128 * (_NS - 1)
        off = pl.multiple_of((_NS - 1) * 128, 8)
        pltpu.sync_copy(xtraj.at[pl.ds(off, tail)], rows_b.at[pl.ds(0, tail)])
        pltpu.sync_copy(rows_b.at[pl.ds(0, tail)], tab_sh.at[pl.ds(off, tail)])

    plsc.subcore_barrier()

    def _scale(j, buf, ebuf):
        def _group(g, carry2):
            a = attr_v[j, pl.ds(g * 16, 16)]
            w = jnp.exp(a)
            ebuf[pl.ds(g * 16, 16)] = w
            for t in range(16):
                ws = jnp.full((16,), w[t])
                e = g * 16 + t
                for k in range(8):
                    sl = pl.ds(k * 16, 16)
                    buf[e, sl] = buf[e, sl] * ws
            return carry2
        lax.fori_loop(0, 8, _group, 0)

    pltpu.async_copy(tab_sh.at[col_v.at[0]], rows_a, sem_a)

    def _pair(p, carry):
        c0 = 2 * p
        pltpu.make_async_copy(tab_sh.at[col_v.at[c0]], rows_a, sem_a).wait()
        pltpu.async_copy(tab_sh.at[col_v.at[c0 + 1]], rows_b, sem_b)
        _scale(c0, rows_a, exp_a)
        pltpu.async_copy(rows_a, agg_sh.at[row_v.at[c0]], sem_sa, add=True)
        pltpu.async_copy(exp_a, ss_sh.at[row_v.at[c0]], sem_ea, add=True)
        pltpu.make_async_copy(tab_sh.at[col_v.at[c0 + 1]], rows_b, sem_b).wait()
        _scale(c0 + 1, rows_b, exp_b)
        pltpu.async_copy(rows_b, agg_sh.at[row_v.at[c0 + 1]], sem_sb, add=True)
        pltpu.async_copy(exp_b, ss_sh.at[row_v.at[c0 + 1]], sem_eb, add=True)
        pltpu.make_async_copy(rows_a, agg_sh.at[row_v.at[c0]], sem_sa).wait()
        c2 = jnp.minimum(c0 + 2, _NCHUNK - 1)
        pltpu.async_copy(tab_sh.at[col_v.at[c2]], rows_a, sem_a)
        pltpu.make_async_copy(rows_b, agg_sh.at[row_v.at[c0 + 1]], sem_sb).wait()
        pltpu.make_async_copy(exp_a, ss_sh.at[row_v.at[c0]], sem_ea).wait()
        pltpu.make_async_copy(exp_b, ss_sh.at[row_v.at[c0 + 1]], sem_eb).wait()
        return carry
    lax.fori_loop(0, _NCHUNK // 2, _pair, 0)
    pltpu.make_async_copy(
        tab_sh.at[col_v.at[_NCHUNK - 1]], rows_a, sem_a).wait()

    plsc.subcore_barrier()

    pltpu.sync_copy(agg_sh.at[pl.ds(s * _RPT, _RPT)],
                    agg_out.at[c, pl.ds(s * _RPT, _RPT)])
    pltpu.sync_copy(ss_sh.at[pl.ds(s * _RPT, _RPT)],
                    ss_out.at[c, pl.ds(s * _RPT, _RPT)])


_sc_edge_call = pl.kernel(
    _sc_edge_body,
    out_type=(
        jax.ShapeDtypeStruct((_NC, _SEG_PAD, _D), jnp.float32),
        jax.ShapeDtypeStruct((_NC, _SEG_PAD), jnp.float32),
    ),
    mesh=plsc.VectorSubcoreMesh(core_axis_name="c", subcore_axis_name="s",
                                num_cores=_NC, num_subcores=_NS),
    scratch_types=[
        pltpu.VMEM_SHARED((_SEG_PAD, _D), jnp.float32),
        pltpu.VMEM_SHARED((_SEG_PAD,), jnp.float32),
        pltpu.VMEM_SHARED((_NUM_TRAJ, _D), jnp.float32),
        pltpu.VMEM((_NCHUNK, _CHUNK), jnp.int32),
        pltpu.VMEM((_NCHUNK, _CHUNK), jnp.int32),
        pltpu.VMEM((_NCHUNK, _CHUNK), jnp.float32),
        pltpu.VMEM((_CHUNK, _D), jnp.float32),
        pltpu.VMEM((_CHUNK, _D), jnp.float32),
        pltpu.VMEM((_CHUNK,), jnp.float32),
        pltpu.VMEM((_CHUNK,), jnp.float32),
        pltpu.VMEM((_RPT,), jnp.float32),
        pltpu.SemaphoreType.DMA,
        pltpu.SemaphoreType.DMA,
        pltpu.SemaphoreType.DMA,
        pltpu.SemaphoreType.DMA,
        pltpu.SemaphoreType.DMA,
        pltpu.SemaphoreType.DMA,
    ],
)


_BM = 1024
_BN = 1024


def _head_body(xci_ref, fuse_ref, agg0_ref, agg1_ref, ss0_ref, ss1_ref,
               g_ref, b_ref, wt_ref, bias_ref, out_ref, ht_scr):
    @pl.when(pl.program_id(1) == 0)
    def _():
        denom = ss0_ref[...] + ss1_ref[...] + 1e-30
        h = xci_ref[...] + (agg0_ref[...] + agg1_ref[...]) / denom
        mu = jnp.mean(h, axis=1, keepdims=True)
        var = jnp.mean((h - mu) * (h - mu), axis=1, keepdims=True)
        h = (h - mu) * jax.lax.rsqrt(var + 1e-5) * g_ref[...] + b_ref[...]
        ht_scr[...] = (h + fuse_ref[...]).T.astype(jnp.bfloat16)

    out_ref[...] = jnp.dot(wt_ref[...], ht_scr[...],
                           preferred_element_type=jnp.float32) + bias_ref[...]


_head_call = pl.pallas_call(
    _head_body,
    grid=((_NUM_CHECKIN + _BM - 1) // _BM, (_NUM_POI + _BN - 1) // _BN),
    in_specs=[
        pl.BlockSpec((_BM, _D), lambda m, n: (m, 0)),
        pl.BlockSpec((_BM, _D), lambda m, n: (m, 0)),
        pl.BlockSpec((_BM, _D), lambda m, n: (m, 0)),
        pl.BlockSpec((_BM, _D), lambda m, n: (m, 0)),
        pl.BlockSpec((_BM, 1), lambda m, n: (m, 0)),
        pl.BlockSpec((_BM, 1), lambda m, n: (m, 0)),
        pl.BlockSpec((1, _D), lambda m, n: (0, 0)),
        pl.BlockSpec((1, _D), lambda m, n: (0, 0)),
        pl.BlockSpec((_BN, _D), lambda m, n: (n, 0)),
        pl.BlockSpec((_BN, 1), lambda m, n: (n, 0)),
    ],
    out_specs=pl.BlockSpec((_BN, _BM), lambda m, n: (n, m)),
    out_shape=jax.ShapeDtypeStruct((_NUM_POI, _NUM_CHECKIN), jnp.float32),
    scratch_shapes=[pltpu.VMEM((_D, _BM), jnp.bfloat16)],
)


def kernel(checkin_feature, delta_ts, delta_ss, ci2traj_attr, traj2traj_attr,
           time_w, time_b, dist_w, dist_b, tw, sw, ln_g, ln_b, W_out, b_out,
           ci2traj_row, ci2traj_col, traj2traj_row, traj2traj_col):
    del traj2traj_attr, traj2traj_row, traj2traj_col

    dts = delta_ts.reshape(_N, 1)
    dss = delta_ss.reshape(_N, 1)
    tws = (time_w.reshape(1, _D), time_b.reshape(1, _D),
           dist_w.reshape(1, _D), dist_b.reshape(1, _D),
           tw.reshape(1, _D), sw.reshape(1, _D))
    xtraj0 = _fusion_traj_call(dts[:_NUM_TRAJ], dss[:_NUM_TRAJ], *tws)
    xci0, fuse_ci = _fusion_ci_call(
        dts[_NUM_TRAJ:], dss[_NUM_TRAJ:], checkin_feature, *tws)

    pad = _E_PAD - _E1
    row_p = jnp.pad(ci2traj_row.astype(jnp.int32), (0, pad)).reshape(
        _NW * _NCHUNK, _CHUNK)
    col_p = jnp.pad(ci2traj_col.astype(jnp.int32), (0, pad)).reshape(
        _NW * _NCHUNK, _CHUNK)
    attr_p = jnp.pad(ci2traj_attr, (0, pad), constant_values=-1e30).reshape(
        _NW * _NCHUNK, _CHUNK)

    agg_pair, ss_pair = _sc_edge_call(xtraj0, row_p, col_p, attr_p)

    out_t = _head_call(
        xci0, fuse_ci,
        agg_pair[0, :_NUM_CHECKIN], agg_pair[1, :_NUM_CHECKIN],
        ss_pair[0, :_NUM_CHECKIN, None], ss_pair[1, :_NUM_CHECKIN, None],
        ln_g.reshape(1, _D), ln_b.reshape(1, _D),
        W_out.T.astype(jnp.bfloat16), b_out.reshape(_NUM_POI, 1))
    return out_t.T


# speedup vs baseline: 24.0699x; 1.0185x over previous
import functools

import jax
import jax.numpy as jnp
from jax import lax
from jax.experimental import pallas as pl
from jax.experimental.pallas import tpu as pltpu
from jax.experimental.pallas import tpu_sc as plsc

_NUM_TRAJ = 2000
_NUM_CHECKIN = 8000
_N = _NUM_TRAJ + _NUM_CHECKIN
_D = 128
_E1 = 160000
_NUM_POI = 5000

_NC = 2
_NS = 16
_NW = _NC * _NS
_CHUNK = 128
_NCHUNK = 40
_EPT = _CHUNK * _NCHUNK
_E_PAD = _EPT * _NW
_SEG_PAD = 8192
_RPT = _SEG_PAD // _NS


def _fast_cos(x):
    q = jnp.round(x * jnp.float32(0.6366197723675814))
    r = x - q * jnp.float32(1.57079601287841796875)
    r = r - q * jnp.float32(3.139164786504813e-7)
    r = r - q * jnp.float32(5.389759655482421e-15)
    r2 = r * r
    s = r + r * r2 * (jnp.float32(-1.6666654611e-1)
                      + r2 * (jnp.float32(8.3321608736e-3)
                              + r2 * jnp.float32(-1.9515295891e-4)))
    c = 1.0 - 0.5 * r2 + r2 * r2 * (jnp.float32(4.166664568298827e-2)
                                    + r2 * (jnp.float32(-1.388731625493765e-3)
                                            + r2 * jnp.float32(2.443315711809948e-5)))
    k = q.astype(jnp.int32)
    odd = (k & 1) == 1
    neg = ((k + (k & 1)) & 2) != 0
    base = jnp.where(odd, s, c)
    return jnp.where(neg, -base, base)


def _fusion_traj_body(dts_ref, dss_ref, timew_ref, timeb_ref, distw_ref,
                      distb_ref, tw_ref, sw_ref, xtraj_ref):
    t_emb = _fast_cos(dts_ref[...] * (1.0 / 3600.0) * timew_ref[...]
                      + timeb_ref[...])
    s_emb = jnp.maximum(dss_ref[...] * distw_ref[...] + distb_ref[...], 0.0)
    fuse = tw_ref[...] * t_emb + sw_ref[...] * s_emb
    xtraj_ref[...] = jnp.maximum(fuse, 0.0)


_fusion_traj_call = pl.pallas_call(
    _fusion_traj_body,
    out_shape=jax.ShapeDtypeStruct((_NUM_TRAJ, _D), jnp.float32),
)


def _fusion_ci_body(dts_ref, dss_ref, ci_ref, timew_ref, timeb_ref, distw_ref,
                    distb_ref, tw_ref, sw_ref, xci_ref, fuse_ref):
    t_emb = _fast_cos(dts_ref[...] * (1.0 / 3600.0) * timew_ref[...]
                      + timeb_ref[...])
    s_emb = jnp.maximum(dss_ref[...] * distw_ref[...] + distb_ref[...], 0.0)
    fuse = tw_ref[...] * t_emb + sw_ref[...] * s_emb
    xci_ref[...] = jnp.maximum(ci_ref[...] + fuse, 0.0)
    fuse_ref[...] = fuse


_fusion_ci_call = pl.pallas_call(
    _fusion_ci_body,
    out_shape=(
        jax.ShapeDtypeStruct((_NUM_CHECKIN, _D), jnp.float32),
        jax.ShapeDtypeStruct((_NUM_CHECKIN, _D), jnp.float32),
    ),
)


def _sc_edge_body(xtraj, rowr, colr, attrr, agg_out, ss_out,
                  agg_sh, ss_sh, tab_sh, row_v, col_v, attr_v, rows_a, rows_b,
                  exp_a, exp_b, zrow_v, sem_a, sem_b, sem_sa, sem_sb,
                  sem_ea, sem_eb):
    c = lax.axis_index("c")
    s = lax.axis_index("s")
    wid = c * _NS + s

    zero16 = jnp.zeros((16,), jnp.float32)

    pltpu.async_copy(rowr.at[pl.ds(wid * _NCHUNK, _NCHUNK)], row_v, sem_a)
    pltpu.async_copy(colr.at[pl.ds(wid * _NCHUNK, _NCHUNK)], col_v, sem_sa)
    pltpu.async_copy(attrr.at[pl.ds(wid * _NCHUNK, _NCHUNK)], attr_v, sem_sb)
    tail = _NUM_TRAJ - 128 * (_NS - 1)

    @pl.when(s < _NS - 1)
    def _load_full():
        off = pl.multiple_of(s * 128, 8)
        pltpu.async_copy(xtraj.at[pl.ds(off, 128)], rows_b, sem_b)

    @pl.when(s == _NS - 1)
    def _load_tail():
        off = pl.multiple_of((_NS - 1) * 128, 8)
        pltpu.async_copy(xtraj.at[pl.ds(off, tail)], rows_b.at[pl.ds(0, tail)],
                         sem_b)

    def _zrow(i, carry):
        for k in range(8):
            rows_a[i, pl.ds(k * 16, 16)] = zero16
        return carry
    lax.fori_loop(0, 128, _zrow, 0)
    for k in range(32):
        zrow_v[pl.ds(k * 16, 16)] = zero16
    for j in range(_RPT // 128):
        pltpu.sync_copy(rows_a, agg_sh.at[pl.ds(s * _RPT + j * 128, 128)])
    pltpu.sync_copy(zrow_v, ss_sh.at[pl.ds(s * _RPT, _RPT)])

    @pl.when(s < _NS - 1)
    def _stage_full():
        off = pl.multiple_of(s * 128, 8)
        pltpu.make_async_copy(xtraj.at[pl.ds(off, 128)], rows_b, sem_b).wait()
        pltpu.sync_copy(rows_b, tab_sh.at[pl.ds(off, 128)])

    @pl.when(s == _NS - 1)
    def _stage_tail():
        off = pl.multiple_of((_NS - 1) * 128, 8)
        pltpu.make_async_copy(xtraj.at[pl.ds(off, tail)],
                              rows_b.at[pl.ds(0, tail)], sem_b).wait()
        pltpu.sync_copy(rows_b.at[pl.ds(0, tail)], tab_sh.at[pl.ds(off, tail)])

    pltpu.make_async_copy(rowr.at[pl.ds(wid * _NCHUNK, _NCHUNK)], row_v,
                          sem_a).wait()
    pltpu.make_async_copy(colr.at[pl.ds(wid * _NCHUNK, _NCHUNK)], col_v,
                          sem_sa).wait()
    pltpu.make_async_copy(attrr.at[pl.ds(wid * _NCHUNK, _NCHUNK)], attr_v,
                          sem_sb).wait()

    plsc.subcore_barrier()

    def _scale(j, buf, ebuf):
        def _group(g, carry2):
            a = attr_v[j, pl.ds(g * 16, 16)]
            w = jnp.exp(a)
            ebuf[pl.ds(g * 16, 16)] = w
            for t in range(16):
                ws = jnp.full((16,), w[t])
                e = g * 16 + t
                for k in range(8):
                    sl = pl.ds(k * 16, 16)
                    buf[e, sl] = buf[e, sl] * ws
            return carry2
        lax.fori_loop(0, 8, _group, 0)

    pltpu.async_copy(tab_sh.at[col_v.at[0]], rows_a, sem_a)

    def _pair(p, carry):
        c0 = 2 * p
        pltpu.make_async_copy(tab_sh.at[col_v.at[c0]], rows_a, sem_a).wait()
        pltpu.async_copy(tab_sh.at[col_v.at[c0 + 1]], rows_b, sem_b)
        _scale(c0, rows_a, exp_a)
        pltpu.async_copy(rows_a, agg_sh.at[row_v.at[c0]], sem_sa, add=True)
        pltpu.async_copy(exp_a, ss_sh.at[row_v.at[c0]], sem_ea, add=True)
        pltpu.make_async_copy(tab_sh.at[col_v.at[c0 + 1]], rows_b, sem_b).wait()
        _scale(c0 + 1, rows_b, exp_b)
        pltpu.async_copy(rows_b, agg_sh.at[row_v.at[c0 + 1]], sem_sb, add=True)
        pltpu.async_copy(exp_b, ss_sh.at[row_v.at[c0 + 1]], sem_eb, add=True)
        pltpu.make_async_copy(rows_a, agg_sh.at[row_v.at[c0]], sem_sa).wait()
        c2 = jnp.minimum(c0 + 2, _NCHUNK - 1)
        pltpu.async_copy(tab_sh.at[col_v.at[c2]], rows_a, sem_a)
        pltpu.make_async_copy(rows_b, agg_sh.at[row_v.at[c0 + 1]], sem_sb).wait()
        pltpu.make_async_copy(exp_a, ss_sh.at[row_v.at[c0]], sem_ea).wait()
        pltpu.make_async_copy(exp_b, ss_sh.at[row_v.at[c0 + 1]], sem_eb).wait()
        return carry
    lax.fori_loop(0, _NCHUNK // 2, _pair, 0)
    pltpu.make_async_copy(
        tab_sh.at[col_v.at[_NCHUNK - 1]], rows_a, sem_a).wait()

    plsc.subcore_barrier()

    pltpu.sync_copy(agg_sh.at[pl.ds(s * _RPT, _RPT)],
                    agg_out.at[c, pl.ds(s * _RPT, _RPT)])
    pltpu.sync_copy(ss_sh.at[pl.ds(s * _RPT, _RPT)],
                    ss_out.at[c, pl.ds(s * _RPT, _RPT)])


_sc_edge_call = pl.kernel(
    _sc_edge_body,
    out_type=(
        jax.ShapeDtypeStruct((_NC, _SEG_PAD, _D), jnp.float32),
        jax.ShapeDtypeStruct((_NC, _SEG_PAD), jnp.float32),
    ),
    mesh=plsc.VectorSubcoreMesh(core_axis_name="c", subcore_axis_name="s",
                                num_cores=_NC, num_subcores=_NS),
    scratch_types=[
        pltpu.VMEM_SHARED((_SEG_PAD, _D), jnp.float32),
        pltpu.VMEM_SHARED((_SEG_PAD,), jnp.float32),
        pltpu.VMEM_SHARED((_NUM_TRAJ, _D), jnp.float32),
        pltpu.VMEM((_NCHUNK, _CHUNK), jnp.int32),
        pltpu.VMEM((_NCHUNK, _CHUNK), jnp.int32),
        pltpu.VMEM((_NCHUNK, _CHUNK), jnp.float32),
        pltpu.VMEM((_CHUNK, _D), jnp.float32),
        pltpu.VMEM((_CHUNK, _D), jnp.float32),
        pltpu.VMEM((_CHUNK,), jnp.float32),
        pltpu.VMEM((_CHUNK,), jnp.float32),
        pltpu.VMEM((_RPT,), jnp.float32),
        pltpu.SemaphoreType.DMA,
        pltpu.SemaphoreType.DMA,
        pltpu.SemaphoreType.DMA,
        pltpu.SemaphoreType.DMA,
        pltpu.SemaphoreType.DMA,
        pltpu.SemaphoreType.DMA,
    ],
)


_BM = 1024
_BN = 1024


def _head_body(xci_ref, fuse_ref, agg0_ref, agg1_ref, ss0_ref, ss1_ref,
               g_ref, b_ref, wt_ref, bias_ref, out_ref, ht_scr):
    @pl.when(pl.program_id(1) == 0)
    def _():
        denom = ss0_ref[...] + ss1_ref[...] + 1e-30
        h = xci_ref[...] + (agg0_ref[...] + agg1_ref[...]) / denom
        mu = jnp.mean(h, axis=1, keepdims=True)
        var = jnp.mean((h - mu) * (h - mu), axis=1, keepdims=True)
        h = (h - mu) * jax.lax.rsqrt(var + 1e-5) * g_ref[...] + b_ref[...]
        ht_scr[...] = (h + fuse_ref[...]).T.astype(jnp.bfloat16)

    out_ref[...] = jnp.dot(wt_ref[...], ht_scr[...],
                           preferred_element_type=jnp.float32) + bias_ref[...]


_head_call = pl.pallas_call(
    _head_body,
    grid=((_NUM_CHECKIN + _BM - 1) // _BM, (_NUM_POI + _BN - 1) // _BN),
    in_specs=[
        pl.BlockSpec((_BM, _D), lambda m, n: (m, 0)),
        pl.BlockSpec((_BM, _D), lambda m, n: (m, 0)),
        pl.BlockSpec((_BM, _D), lambda m, n: (m, 0)),
        pl.BlockSpec((_BM, _D), lambda m, n: (m, 0)),
        pl.BlockSpec((_BM, 1), lambda m, n: (m, 0)),
        pl.BlockSpec((_BM, 1), lambda m, n: (m, 0)),
        pl.BlockSpec((1, _D), lambda m, n: (0, 0)),
        pl.BlockSpec((1, _D), lambda m, n: (0, 0)),
        pl.BlockSpec((_BN, _D), lambda m, n: (n, 0)),
        pl.BlockSpec((_BN, 1), lambda m, n: (n, 0)),
    ],
    out_specs=pl.BlockSpec((_BN, _BM), lambda m, n: (n, m)),
    out_shape=jax.ShapeDtypeStruct((_NUM_POI, _NUM_CHECKIN), jnp.float32),
    scratch_shapes=[pltpu.VMEM((_D, _BM), jnp.bfloat16)],
)


def kernel(checkin_feature, delta_ts, delta_ss, ci2traj_attr, traj2traj_attr,
           time_w, time_b, dist_w, dist_b, tw, sw, ln_g, ln_b, W_out, b_out,
           ci2traj_row, ci2traj_col, traj2traj_row, traj2traj_col):
    del traj2traj_attr, traj2traj_row, traj2traj_col

    dts = delta_ts.reshape(_N, 1)
    dss = delta_ss.reshape(_N, 1)
    tws = (time_w.reshape(1, _D), time_b.reshape(1, _D),
           dist_w.reshape(1, _D), dist_b.reshape(1, _D),
           tw.reshape(1, _D), sw.reshape(1, _D))
    xtraj0 = _fusion_traj_call(dts[:_NUM_TRAJ], dss[:_NUM_TRAJ], *tws)
    xci0, fuse_ci = _fusion_ci_call(
        dts[_NUM_TRAJ:], dss[_NUM_TRAJ:], checkin_feature, *tws)

    pad = _E_PAD - _E1
    row_p = jnp.pad(ci2traj_row.astype(jnp.int32), (0, pad)).reshape(
        _NW * _NCHUNK, _CHUNK)
    col_p = jnp.pad(ci2traj_col.astype(jnp.int32), (0, pad)).reshape(
        _NW * _NCHUNK, _CHUNK)
    attr_p = jnp.pad(ci2traj_attr, (0, pad), constant_values=-1e30).reshape(
        _NW * _NCHUNK, _CHUNK)

    agg_pair, ss_pair = _sc_edge_call(xtraj0, row_p, col_p, attr_p)

    out_t = _head_call(
        xci0, fuse_ci,
        agg_pair[0, :_NUM_CHECKIN], agg_pair[1, :_NUM_CHECKIN],
        ss_pair[0, :_NUM_CHECKIN, None], ss_pair[1, :_NUM_CHECKIN, None],
        ln_g.reshape(1, _D), ln_b.reshape(1, _D),
        W_out.T.astype(jnp.bfloat16), b_out.reshape(_NUM_POI, 1))
    return out_t.T
